# baseline (speedup 1.0000x reference)
import functools

import jax
import jax.numpy as jnp
import numpy as np
from jax import lax
from jax.experimental import pallas as pl
from jax.experimental.pallas import tpu as pltpu
from jax.experimental.pallas import tpu_sc as plsc

B = 4
P = 50
H = 200
W = 200
C = 80
R = 2048
S = 128

NC = 2
NS = 16
NW = NC * NS
RPW = R // NW
GROUPS = RPW // 16
GSZ = 16 * S

PLANE = 400 * 128
HALF = 200 * 128

INV_T = np.float32(1.0 / (S - 1))
INV_S = np.float32(1.0 / S)
MAGIC = np.float32(2.0 ** 23)
RSQRT_MAGIC = np.int32(0x5F3759DF)
LOG_EPS = np.float32(np.log(np.float32(1e-12)))
NEG_INF = np.float32(-np.inf)
LOSS_W = np.float32(0.1)


def _raf_pack_body(c0_ref, c1_ref, out_ref):
    c0 = jnp.minimum(jnp.maximum(c0_ref[0, 0], np.float32(-1.0)), np.float32(1.0))
    c1 = jnp.minimum(jnp.maximum(c1_ref[0, 0], np.float32(-1.0)), np.float32(1.0))
    b0 = lax.bitcast_convert_type(c0.astype(jnp.bfloat16), jnp.uint16)
    b1 = lax.bitcast_convert_type(c1.astype(jnp.bfloat16), jnp.uint16)
    u = jnp.left_shift(b1.astype(jnp.int32), 16) | b0.astype(jnp.int32)
    out_ref[pl.ds(0, 200), :] = u[:, 0:128]
    out_ref[pl.ds(200, 200), :] = u[:, 72:200]


_raf_pack = pl.pallas_call(
    _raf_pack_body,
    grid=(B * P,),
    in_specs=[
        pl.BlockSpec((1, 1, H, W), lambda i: (i // P, 2 * (i % P), 0, 0)),
        pl.BlockSpec((1, 1, H, W), lambda i: (i // P, 2 * (i % P) + 1, 0, 0)),
    ],
    out_specs=pl.BlockSpec((400, 128), lambda i: (i, 0)),
    out_shape=jax.ShapeDtypeStruct((B * P * 400, 128), jnp.int32),
)


def _hm_pack_body(in_ref, out_ref):
    v = in_ref[0, 0]
    out_ref[pl.ds(0, 200), :] = v[:, 0:128]
    out_ref[pl.ds(200, 200), :] = v[:, 72:200]


_hm_pack = pl.pallas_call(
    _hm_pack_body,
    grid=(B * C,),
    in_specs=[pl.BlockSpec((1, 1, H, W), lambda i: (i // C, i % C, 0, 0))],
    out_specs=pl.BlockSpec((400, 128), lambda i: (i, 0)),
    out_shape=jax.ShapeDtypeStruct((B * C * 400, 128), jnp.float32),
)


def _rsqrt_f32(x):
    i = lax.bitcast_convert_type(x, jnp.int32)
    i = RSQRT_MAGIC - lax.shift_right_logical(i, 1)
    y = lax.bitcast_convert_type(i, jnp.float32)
    for _ in range(3):
        y = y * (np.float32(1.5) - np.float32(0.5) * x * y * y)
    return y


def _rint_idx(x):
    r = (x + MAGIC) - MAGIC
    r = jnp.minimum(jnp.maximum(r, np.float32(0.0)), np.float32(199.0))
    return r.astype(jnp.int32)


def _lane_idx(p, q):
    return q * 128 + jnp.where(p < 128, p, HALF - 72 + p)


def _sc_body(raf_hbm, hm_hbm, bi_hbm, scl_hbm, ocl_hbm, prd_hbm,
             sx_hbm, sy_hbm, ox_hbm, oy_hbm,
             integ_hbm, so_hbm, valid_hbm,
             int_buf, idx_buf, g_buf, hidx, hval,
             uxb, uyb, integb, sob, validb, sem_in, sem_r, sem_h):
    wid = lax.axis_index("c") * NS + lax.axis_index("s")
    base = wid * RPW

    ins = (bi_hbm, scl_hbm, ocl_hbm, prd_hbm, sx_hbm, sy_hbm, ox_hbm, oy_hbm)
    cps = [pltpu.async_copy(src.at[pl.ds(base, RPW)],
                            int_buf.at[pl.ds(f * RPW, RPW)], sem_in)
           for f, src in enumerate(ins)]
    for cp in cps:
        cp.wait()

    def meta_group(g, _):
        off = g * 16
        bi = int_buf[pl.ds(0 * RPW + off, 16)]
        scl = int_buf[pl.ds(1 * RPW + off, 16)]
        ocl = int_buf[pl.ds(2 * RPW + off, 16)]
        sxi = int_buf[pl.ds(4 * RPW + off, 16)]
        syi = int_buf[pl.ds(5 * RPW + off, 16)]
        oxi = int_buf[pl.ds(6 * RPW + off, 16)]
        oyi = int_buf[pl.ds(7 * RPW + off, 16)]

        hidx[pl.ds(off, 16)] = (bi * C + scl) * PLANE + _lane_idx(sxi, syi)
        hidx[pl.ds(RPW + off, 16)] = (bi * C + ocl) * PLANE + _lane_idx(oxi, oyi)

        sxf = sxi.astype(jnp.float32)
        syf = syi.astype(jnp.float32)
        oxf = oxi.astype(jnp.float32)
        oyf = oyi.astype(jnp.float32)
        dx = oxf - sxf
        dy = oyf - syf
        n2 = dx * dx + dy * dy
        r = _rsqrt_f32(jnp.maximum(n2, np.float32(1.0)))
        uxb[pl.ds(off, 16)] = dx * r
        uyb[pl.ds(off, 16)] = dy * r
        validb[pl.ds(off, 16)] = jnp.where(n2 > np.float32(0.0),
                                           np.float32(1.0), np.float32(0.0))
        return 0

    lax.fori_loop(0, GROUPS, meta_group, 0)
    cp_h = pltpu.async_copy(hm_hbm.at[hidx], hval, sem_h)

    def build_group(g):
        off = g * 16
        bi = int_buf[pl.ds(0 * RPW + off, 16)]
        prd = int_buf[pl.ds(3 * RPW + off, 16)]
        sxf = int_buf[pl.ds(4 * RPW + off, 16)].astype(jnp.float32)
        syf = int_buf[pl.ds(5 * RPW + off, 16)].astype(jnp.float32)
        oxf = int_buf[pl.ds(6 * RPW + off, 16)].astype(jnp.float32)
        oyf = int_buf[pl.ds(7 * RPW + off, 16)].astype(jnp.float32)
        rbase = (bi * P + prd) * PLANE
        ddx = sxf - oxf
        ddy = syf - oyf

        def build_row(r32, _):
            rowoff = g * GSZ + r32 * 64
            for k in range(4):
                s = r32 * 4 + k
                t = s.astype(jnp.float32) * INV_T
                px = _rint_idx(oxf + t * ddx)
                py = _rint_idx(oyf + t * ddy)
                idx_buf[pl.ds(rowoff + k * 16, 16)] = rbase + _lane_idx(px, py)
            return 0

        lax.fori_loop(0, 32, build_row, 0)

    def reduce_group(g):
        off = g * 16
        ux = uxb[pl.ds(off, 16)]
        uy = uyb[pl.ds(off, 16)]

        def red_row(r32, acc):
            rowoff = g * GSZ + r32 * 64
            for k in range(4):
                v = g_buf[pl.ds(rowoff + k * 16, 16)]
                g1 = lax.bitcast_convert_type(
                    v & np.int32(-65536), jnp.float32)
                g0 = lax.bitcast_convert_type(
                    jnp.left_shift(v, 16), jnp.float32)
                acc = acc + g0 * ux + g1 * uy
            return acc

        acc = lax.fori_loop(0, 32, red_row, jnp.zeros((16,), jnp.float32))
        integb[pl.ds(off, 16)] = jnp.minimum(
            jnp.maximum(acc * INV_S, np.float32(0.0)), np.float32(1.0))

    raf_cps = []
    for g in range(GROUPS):
        build_group(g)
        raf_cps.append(pltpu.async_copy(
            raf_hbm.at[idx_buf.at[pl.ds(g * GSZ, GSZ)]],
            g_buf.at[pl.ds(g * GSZ, GSZ)], sem_r))

    cp_h.wait()

    def so_group(g, _):
        off = g * 16
        sob[pl.ds(off, 16)] = hval[pl.ds(off, 16)] * hval[pl.ds(RPW + off, 16)]
        return 0

    lax.fori_loop(0, GROUPS, so_group, 0)

    for g in range(GROUPS):
        raf_cps[g].wait()
        reduce_group(g)

    pltpu.sync_copy(integb, integ_hbm.at[pl.ds(base, RPW)])
    pltpu.sync_copy(sob, so_hbm.at[pl.ds(base, RPW)])
    pltpu.sync_copy(validb, valid_hbm.at[pl.ds(base, RPW)])


_sc_compute = functools.partial(
    pl.kernel,
    out_type=(jax.ShapeDtypeStruct((R,), jnp.float32),
              jax.ShapeDtypeStruct((R,), jnp.float32),
              jax.ShapeDtypeStruct((R,), jnp.float32)),
    mesh=plsc.VectorSubcoreMesh(core_axis_name="c", subcore_axis_name="s",
                                num_cores=NC, num_subcores=NS),
    scratch_types=[
        pltpu.VMEM((8 * RPW,), jnp.int32),
        pltpu.VMEM((8192,), jnp.int32),
        pltpu.VMEM((8192,), jnp.int32),
        pltpu.VMEM((2 * RPW,), jnp.int32),
        pltpu.VMEM((2 * RPW,), jnp.float32),
        pltpu.VMEM((RPW,), jnp.float32),
        pltpu.VMEM((RPW,), jnp.float32),
        pltpu.VMEM((RPW,), jnp.float32),
        pltpu.VMEM((RPW,), jnp.float32),
        pltpu.VMEM((RPW,), jnp.float32),
        pltpu.SemaphoreType.DMA,
        pltpu.SemaphoreType.DMA,
        pltpu.SemaphoreType.DMA,
    ],
)(_sc_body)


def _loss_body(so_col, integ_row, valid_col, valid_row, out_ref):
    mj = valid_row[...]
    b_row = jnp.where(mj > np.float32(0.0),
                      jnp.log(integ_row[...]), NEG_INF)

    def body(i, acc):
        so16 = so_col[pl.ds(i * 16, 16), :]
        mi = valid_col[pl.ds(i * 16, 16), :]
        a16 = jnp.where(mi > np.float32(0.0),
                        jnp.log(so16) - LOG_EPS, NEG_INF)
        return acc + jnp.maximum(a16 + b_row, np.float32(0.0))

    acc = lax.fori_loop(0, R // 16, body,
                        jnp.zeros((16, R), jnp.float32))
    nv = jnp.sum(mj)
    s = jnp.sum(acc) + LOG_EPS * nv * nv
    loss = -s / jnp.maximum(nv * nv, np.float32(1.0)) * LOSS_W
    out_ref[...] = loss.reshape(1, 1)


_loss_call = pl.pallas_call(
    _loss_body,
    out_shape=jax.ShapeDtypeStruct((1, 1), jnp.float32),
)


def kernel(rafs, heatmaps, batch_inds, subj_classes, obj_classes,
           subj_centers, obj_centers, predicates):
    raf_flat = _raf_pack(rafs, rafs).reshape(-1)
    hm_flat = _hm_pack(heatmaps).reshape(-1)
    integ, so, valid = _sc_compute(
        raf_flat, hm_flat, batch_inds, subj_classes, obj_classes, predicates,
        subj_centers[:, 0], subj_centers[:, 1],
        obj_centers[:, 0], obj_centers[:, 1])
    loss = _loss_call(so.reshape(R, 1), integ.reshape(1, R),
                      valid.reshape(R, 1), valid.reshape(1, R))
    return loss.reshape(())


# speedup vs baseline: 2.3936x; 2.3936x over previous
import functools

import jax
import jax.numpy as jnp
import numpy as np
from jax import lax
from jax.experimental import pallas as pl
from jax.experimental.pallas import tpu as pltpu
from jax.experimental.pallas import tpu_sc as plsc

B = 4
P = 50
H = 200
W = 200
C = 80
R = 2048
S = 128

NC = 2
NS = 16
NW = NC * NS
RPW = R // NW
GROUPS = RPW // 16
GSZ = 16 * S

PLANE = 400 * 128
HALF = 200 * 128

INV_T = np.float32(1.0 / (S - 1))
INV_S = np.float32(1.0 / S)
MAGIC = np.float32(2.0 ** 23)
RSQRT_MAGIC = np.int32(0x5F3759DF)
LOG_EPS = np.float32(np.log(np.float32(1e-12)))
NEG_INF = np.float32(-np.inf)
LOSS_W = np.float32(0.1)


RPP = 5
HPP = 8


def _raf_pack_body(in_ref, out_ref):
    for j in range(RPP):
        c0 = jnp.minimum(jnp.maximum(in_ref[0, 2 * j], np.float32(-1.0)),
                         np.float32(1.0))
        c1 = jnp.minimum(jnp.maximum(in_ref[0, 2 * j + 1], np.float32(-1.0)),
                         np.float32(1.0))
        b0 = lax.bitcast_convert_type(c0.astype(jnp.bfloat16), jnp.uint16)
        b1 = lax.bitcast_convert_type(c1.astype(jnp.bfloat16), jnp.uint16)
        u = jnp.left_shift(b1.astype(jnp.int32), 16) | b0.astype(jnp.int32)
        out_ref[pl.ds(j * 400, 200), :] = u[:, 0:128]
        out_ref[pl.ds(j * 400 + 200, 200), :] = u[:, 72:200]


_raf_pack = pl.pallas_call(
    _raf_pack_body,
    grid=(B * P // RPP,),
    in_specs=[
        pl.BlockSpec((1, 2 * RPP, H, W),
                     lambda i: (i // (P // RPP), i % (P // RPP), 0, 0)),
    ],
    out_specs=pl.BlockSpec((400 * RPP, 128), lambda i: (i, 0)),
    out_shape=jax.ShapeDtypeStruct((B * P * 400, 128), jnp.int32),
)


def _hm_pack_body(in_ref, out_ref):
    for j in range(HPP):
        v = in_ref[0, j]
        out_ref[pl.ds(j * 400, 200), :] = v[:, 0:128]
        out_ref[pl.ds(j * 400 + 200, 200), :] = v[:, 72:200]


_hm_pack = pl.pallas_call(
    _hm_pack_body,
    grid=(B * C // HPP,),
    in_specs=[
        pl.BlockSpec((1, HPP, H, W),
                     lambda i: (i // (C // HPP), i % (C // HPP), 0, 0)),
    ],
    out_specs=pl.BlockSpec((400 * HPP, 128), lambda i: (i, 0)),
    out_shape=jax.ShapeDtypeStruct((B * C * 400, 128), jnp.float32),
)


def _rsqrt_f32(x):
    i = lax.bitcast_convert_type(x, jnp.int32)
    i = RSQRT_MAGIC - lax.shift_right_logical(i, 1)
    y = lax.bitcast_convert_type(i, jnp.float32)
    for _ in range(3):
        y = y * (np.float32(1.5) - np.float32(0.5) * x * y * y)
    return y


def _rint_idx(x):
    r = (x + MAGIC) - MAGIC
    r = jnp.minimum(jnp.maximum(r, np.float32(0.0)), np.float32(199.0))
    return r.astype(jnp.int32)


def _lane_idx(p, q):
    return q * 128 + jnp.where(p < 128, p, HALF - 72 + p)


def _sc_body(raf_hbm, hm_hbm, bi_hbm, scl_hbm, ocl_hbm, prd_hbm,
             sx_hbm, sy_hbm, ox_hbm, oy_hbm,
             integ_hbm, so_hbm, valid_hbm,
             int_buf, idx_buf, g_buf, hidx, hval,
             uxb, uyb, integb, sob, validb, sem_in, sem_r, sem_h):
    wid = lax.axis_index("c") * NS + lax.axis_index("s")
    base = wid * RPW

    ins = (bi_hbm, scl_hbm, ocl_hbm, prd_hbm, sx_hbm, sy_hbm, ox_hbm, oy_hbm)
    cps = [pltpu.async_copy(src.at[pl.ds(base, RPW)],
                            int_buf.at[pl.ds(f * RPW, RPW)], sem_in)
           for f, src in enumerate(ins)]
    for cp in cps:
        cp.wait()

    def meta_group(g, _):
        off = g * 16
        bi = int_buf[pl.ds(0 * RPW + off, 16)]
        scl = int_buf[pl.ds(1 * RPW + off, 16)]
        ocl = int_buf[pl.ds(2 * RPW + off, 16)]
        sxi = int_buf[pl.ds(4 * RPW + off, 16)]
        syi = int_buf[pl.ds(5 * RPW + off, 16)]
        oxi = int_buf[pl.ds(6 * RPW + off, 16)]
        oyi = int_buf[pl.ds(7 * RPW + off, 16)]

        hidx[pl.ds(off, 16)] = (bi * C + scl) * PLANE + _lane_idx(sxi, syi)
        hidx[pl.ds(RPW + off, 16)] = (bi * C + ocl) * PLANE + _lane_idx(oxi, oyi)

        sxf = sxi.astype(jnp.float32)
        syf = syi.astype(jnp.float32)
        oxf = oxi.astype(jnp.float32)
        oyf = oyi.astype(jnp.float32)
        dx = oxf - sxf
        dy = oyf - syf
        n2 = dx * dx + dy * dy
        r = _rsqrt_f32(jnp.maximum(n2, np.float32(1.0)))
        uxb[pl.ds(off, 16)] = dx * r
        uyb[pl.ds(off, 16)] = dy * r
        validb[pl.ds(off, 16)] = jnp.where(n2 > np.float32(0.0),
                                           np.float32(1.0), np.float32(0.0))
        return 0

    lax.fori_loop(0, GROUPS, meta_group, 0)
    cp_h = pltpu.async_copy(hm_hbm.at[hidx], hval, sem_h)

    def build_group(g):
        off = g * 16
        bi = int_buf[pl.ds(0 * RPW + off, 16)]
        prd = int_buf[pl.ds(3 * RPW + off, 16)]
        sxf = int_buf[pl.ds(4 * RPW + off, 16)].astype(jnp.float32)
        syf = int_buf[pl.ds(5 * RPW + off, 16)].astype(jnp.float32)
        oxf = int_buf[pl.ds(6 * RPW + off, 16)].astype(jnp.float32)
        oyf = int_buf[pl.ds(7 * RPW + off, 16)].astype(jnp.float32)
        rbase = (bi * P + prd) * PLANE
        ddx = sxf - oxf
        ddy = syf - oyf

        def build_row(r32, _):
            rowoff = g * GSZ + r32 * 64
            for k in range(4):
                s = r32 * 4 + k
                t = s.astype(jnp.float32) * INV_T
                px = _rint_idx(oxf + t * ddx)
                py = _rint_idx(oyf + t * ddy)
                idx_buf[pl.ds(rowoff + k * 16, 16)] = rbase + _lane_idx(px, py)
            return 0

        lax.fori_loop(0, 32, build_row, 0)

    def reduce_group(g):
        off = g * 16
        ux = uxb[pl.ds(off, 16)]
        uy = uyb[pl.ds(off, 16)]

        def red_row(r32, acc):
            rowoff = g * GSZ + r32 * 64
            for k in range(4):
                v = g_buf[pl.ds(rowoff + k * 16, 16)]
                g1 = lax.bitcast_convert_type(
                    v & np.int32(-65536), jnp.float32)
                g0 = lax.bitcast_convert_type(
                    jnp.left_shift(v, 16), jnp.float32)
                acc = acc + g0 * ux + g1 * uy
            return acc

        acc = lax.fori_loop(0, 32, red_row, jnp.zeros((16,), jnp.float32))
        integb[pl.ds(off, 16)] = jnp.minimum(
            jnp.maximum(acc * INV_S, np.float32(0.0)), np.float32(1.0))

    raf_cps = []
    for g in range(GROUPS):
        build_group(g)
        raf_cps.append(pltpu.async_copy(
            raf_hbm.at[idx_buf.at[pl.ds(g * GSZ, GSZ)]],
            g_buf.at[pl.ds(g * GSZ, GSZ)], sem_r))

    cp_h.wait()

    def so_group(g, _):
        off = g * 16
        sob[pl.ds(off, 16)] = hval[pl.ds(off, 16)] * hval[pl.ds(RPW + off, 16)]
        return 0

    lax.fori_loop(0, GROUPS, so_group, 0)

    for g in range(GROUPS):
        raf_cps[g].wait()
        reduce_group(g)

    pltpu.sync_copy(integb, integ_hbm.at[pl.ds(base, RPW)])
    pltpu.sync_copy(sob, so_hbm.at[pl.ds(base, RPW)])
    pltpu.sync_copy(validb, valid_hbm.at[pl.ds(base, RPW)])


_sc_compute = functools.partial(
    pl.kernel,
    out_type=(jax.ShapeDtypeStruct((R,), jnp.float32),
              jax.ShapeDtypeStruct((R,), jnp.float32),
              jax.ShapeDtypeStruct((R,), jnp.float32)),
    mesh=plsc.VectorSubcoreMesh(core_axis_name="c", subcore_axis_name="s",
                                num_cores=NC, num_subcores=NS),
    scratch_types=[
        pltpu.VMEM((8 * RPW,), jnp.int32),
        pltpu.VMEM((8192,), jnp.int32),
        pltpu.VMEM((8192,), jnp.int32),
        pltpu.VMEM((2 * RPW,), jnp.int32),
        pltpu.VMEM((2 * RPW,), jnp.float32),
        pltpu.VMEM((RPW,), jnp.float32),
        pltpu.VMEM((RPW,), jnp.float32),
        pltpu.VMEM((RPW,), jnp.float32),
        pltpu.VMEM((RPW,), jnp.float32),
        pltpu.VMEM((RPW,), jnp.float32),
        pltpu.SemaphoreType.DMA,
        pltpu.SemaphoreType.DMA,
        pltpu.SemaphoreType.DMA,
    ],
)(_sc_body)


def _loss_body(so_col, integ_row, valid_col, valid_row, out_ref):
    mj = valid_row[...]
    b_row = jnp.where(mj > np.float32(0.0),
                      jnp.log(integ_row[...]), NEG_INF)

    def body(i, acc):
        so16 = so_col[pl.ds(i * 16, 16), :]
        mi = valid_col[pl.ds(i * 16, 16), :]
        a16 = jnp.where(mi > np.float32(0.0),
                        jnp.log(so16) - LOG_EPS, NEG_INF)
        return acc + jnp.maximum(a16 + b_row, np.float32(0.0))

    acc = lax.fori_loop(0, R // 16, body,
                        jnp.zeros((16, R), jnp.float32))
    nv = jnp.sum(mj)
    s = jnp.sum(acc) + LOG_EPS * nv * nv
    loss = -s / jnp.maximum(nv * nv, np.float32(1.0)) * LOSS_W
    out_ref[...] = loss.reshape(1, 1)


_loss_call = pl.pallas_call(
    _loss_body,
    out_shape=jax.ShapeDtypeStruct((1, 1), jnp.float32),
)


def kernel(rafs, heatmaps, batch_inds, subj_classes, obj_classes,
           subj_centers, obj_centers, predicates):
    raf_flat = _raf_pack(rafs).reshape(-1)
    hm_flat = _hm_pack(heatmaps).reshape(-1)
    integ, so, valid = _sc_compute(
        raf_flat, hm_flat, batch_inds, subj_classes, obj_classes, predicates,
        subj_centers[:, 0], subj_centers[:, 1],
        obj_centers[:, 0], obj_centers[:, 1])
    loss = _loss_call(so.reshape(R, 1), integ.reshape(1, R),
                      valid.reshape(R, 1), valid.reshape(1, R))
    return loss.reshape(())


# speedup vs baseline: 2.5677x; 1.0727x over previous
import functools

import jax
import jax.numpy as jnp
import numpy as np
from jax import lax
from jax.experimental import pallas as pl
from jax.experimental.pallas import tpu as pltpu
from jax.experimental.pallas import tpu_sc as plsc

B = 4
P = 50
H = 200
W = 200
C = 80
R = 2048
S = 128

NC = 2
NS = 16
NW = NC * NS
RPW = R // NW
GROUPS = RPW // 16
GSZ = 16 * S

PLANE = 400 * 128
HALF = 200 * 128

INV_T = np.float32(1.0 / (S - 1))
INV_S = np.float32(1.0 / S)
MAGIC = np.float32(2.0 ** 23)
RSQRT_MAGIC = np.int32(0x5F3759DF)
LOG_EPS = np.float32(np.log(np.float32(1e-12)))
NEG_INF = np.float32(-np.inf)
LOSS_W = np.float32(0.1)


RPP = 10
HPP = 16
HPLANE = 200 * 128
HHALF = 100 * 128


def _raf_pack_body(in_ref, out_ref):
    for j in range(RPP):
        c0 = jnp.minimum(jnp.maximum(in_ref[0, 2 * j], np.float32(-1.0)),
                         np.float32(1.0))
        c1 = jnp.minimum(jnp.maximum(in_ref[0, 2 * j + 1], np.float32(-1.0)),
                         np.float32(1.0))
        b0 = lax.bitcast_convert_type(c0.astype(jnp.bfloat16), jnp.uint16)
        b1 = lax.bitcast_convert_type(c1.astype(jnp.bfloat16), jnp.uint16)
        u = jnp.left_shift(b1.astype(jnp.int32), 16) | b0.astype(jnp.int32)
        out_ref[pl.ds(j * 400, 200), :] = u[:, 0:128]
        out_ref[pl.ds(j * 400 + 200, 200), :] = u[:, 72:200]


_raf_pack = pl.pallas_call(
    _raf_pack_body,
    grid=(B * P // RPP,),
    in_specs=[
        pl.BlockSpec((1, 2 * RPP, H, W),
                     lambda i: (i // (P // RPP), i % (P // RPP), 0, 0)),
    ],
    out_specs=pl.BlockSpec((400 * RPP, 128), lambda i: (i, 0)),
    out_shape=jax.ShapeDtypeStruct((B * P * 400, 128), jnp.int32),
)


def _hm_pack_body(in_ref, out_ref):
    for j in range(HPP):
        v = in_ref[0, j]
        bb = lax.bitcast_convert_type(v.astype(jnp.bfloat16), jnp.uint16)
        b3 = bb.reshape(100, 2, 200)
        be = b3[:, 0, :].astype(jnp.int32)
        bo = b3[:, 1, :].astype(jnp.int32)
        u = jnp.left_shift(bo, 16) | be
        out_ref[pl.ds(j * 200, 100), :] = u[:, 0:128]
        out_ref[pl.ds(j * 200 + 100, 100), :] = u[:, 72:200]


_hm_pack = pl.pallas_call(
    _hm_pack_body,
    grid=(B * C // HPP,),
    in_specs=[
        pl.BlockSpec((1, HPP, H, W),
                     lambda i: (i // (C // HPP), i % (C // HPP), 0, 0)),
    ],
    out_specs=pl.BlockSpec((200 * HPP, 128), lambda i: (i, 0)),
    out_shape=jax.ShapeDtypeStruct((B * C * 200, 128), jnp.int32),
)


def _rsqrt_f32(x):
    i = lax.bitcast_convert_type(x, jnp.int32)
    i = RSQRT_MAGIC - lax.shift_right_logical(i, 1)
    y = lax.bitcast_convert_type(i, jnp.float32)
    for _ in range(3):
        y = y * (np.float32(1.5) - np.float32(0.5) * x * y * y)
    return y


def _rint_idx(x):
    r = (x + MAGIC) - MAGIC
    r = jnp.minimum(jnp.maximum(r, np.float32(0.0)), np.float32(199.0))
    return r.astype(jnp.int32)


def _lane_idx(p, q):
    return q * 128 + jnp.where(p < 128, p, HALF - 72 + p)


def _sc_body(raf_hbm, hm_hbm, bi_hbm, scl_hbm, ocl_hbm, prd_hbm,
             sx_hbm, sy_hbm, ox_hbm, oy_hbm,
             integ_hbm, so_hbm, valid_hbm,
             int_buf, idx_buf, g_buf, hidx, hval,
             uxb, uyb, integb, sob, validb, sem_in, sem_r, sem_h):
    wid = lax.axis_index("c") * NS + lax.axis_index("s")
    base = wid * RPW

    ins = (bi_hbm, scl_hbm, ocl_hbm, prd_hbm, sx_hbm, sy_hbm, ox_hbm, oy_hbm)
    cps = [pltpu.async_copy(src.at[pl.ds(base, RPW)],
                            int_buf.at[pl.ds(f * RPW, RPW)], sem_in)
           for f, src in enumerate(ins)]
    for cp in cps:
        cp.wait()

    def meta_group(g, _):
        off = g * 16
        bi = int_buf[pl.ds(0 * RPW + off, 16)]
        scl = int_buf[pl.ds(1 * RPW + off, 16)]
        ocl = int_buf[pl.ds(2 * RPW + off, 16)]
        sxi = int_buf[pl.ds(4 * RPW + off, 16)]
        syi = int_buf[pl.ds(5 * RPW + off, 16)]
        oxi = int_buf[pl.ds(6 * RPW + off, 16)]
        oyi = int_buf[pl.ds(7 * RPW + off, 16)]

        hidx[pl.ds(off, 16)] = ((bi * C + scl) * HPLANE +
                                lax.shift_right_logical(syi, 1) * 128 +
                                jnp.where(sxi < 128, sxi, HHALF - 72 + sxi))
        hidx[pl.ds(RPW + off, 16)] = ((bi * C + ocl) * HPLANE +
                                      lax.shift_right_logical(oyi, 1) * 128 +
                                      jnp.where(oxi < 128, oxi, HHALF - 72 + oxi))

        sxf = sxi.astype(jnp.float32)
        syf = syi.astype(jnp.float32)
        oxf = oxi.astype(jnp.float32)
        oyf = oyi.astype(jnp.float32)
        dx = oxf - sxf
        dy = oyf - syf
        n2 = dx * dx + dy * dy
        r = _rsqrt_f32(jnp.maximum(n2, np.float32(1.0)))
        uxb[pl.ds(off, 16)] = dx * r
        uyb[pl.ds(off, 16)] = dy * r
        validb[pl.ds(off, 16)] = jnp.where(n2 > np.float32(0.0),
                                           np.float32(1.0), np.float32(0.0))
        return 0

    lax.fori_loop(0, GROUPS, meta_group, 0)
    cp_h = pltpu.async_copy(hm_hbm.at[hidx], hval, sem_h)

    def build_group(g):
        off = g * 16
        bi = int_buf[pl.ds(0 * RPW + off, 16)]
        prd = int_buf[pl.ds(3 * RPW + off, 16)]
        sxf = int_buf[pl.ds(4 * RPW + off, 16)].astype(jnp.float32)
        syf = int_buf[pl.ds(5 * RPW + off, 16)].astype(jnp.float32)
        oxf = int_buf[pl.ds(6 * RPW + off, 16)].astype(jnp.float32)
        oyf = int_buf[pl.ds(7 * RPW + off, 16)].astype(jnp.float32)
        rbase = (bi * P + prd) * PLANE
        ddx = sxf - oxf
        ddy = syf - oyf

        def build_row(r32, _):
            rowoff = g * GSZ + r32 * 64
            for k in range(4):
                s = r32 * 4 + k
                t = s.astype(jnp.float32) * INV_T
                px = _rint_idx(oxf + t * ddx)
                py = _rint_idx(oyf + t * ddy)
                idx_buf[pl.ds(rowoff + k * 16, 16)] = rbase + _lane_idx(px, py)
            return 0

        lax.fori_loop(0, 32, build_row, 0)

    def reduce_group(g):
        off = g * 16
        ux = uxb[pl.ds(off, 16)]
        uy = uyb[pl.ds(off, 16)]

        def red_row(r32, acc):
            rowoff = g * GSZ + r32 * 64
            for k in range(4):
                v = g_buf[pl.ds(rowoff + k * 16, 16)]
                g1 = lax.bitcast_convert_type(
                    v & np.int32(-65536), jnp.float32)
                g0 = lax.bitcast_convert_type(
                    jnp.left_shift(v, 16), jnp.float32)
                acc = acc + g0 * ux + g1 * uy
            return acc

        acc = lax.fori_loop(0, 32, red_row, jnp.zeros((16,), jnp.float32))
        integb[pl.ds(off, 16)] = jnp.minimum(
            jnp.maximum(acc * INV_S, np.float32(0.0)), np.float32(1.0))

    raf_cps = []
    for g in range(GROUPS):
        build_group(g)
        raf_cps.append(pltpu.async_copy(
            raf_hbm.at[idx_buf.at[pl.ds(g * GSZ, GSZ)]],
            g_buf.at[pl.ds(g * GSZ, GSZ)], sem_r))

    cp_h.wait()

    def so_group(g, _):
        off = g * 16
        syi = int_buf[pl.ds(5 * RPW + off, 16)]
        oyi = int_buf[pl.ds(7 * RPW + off, 16)]
        ws = hval[pl.ds(off, 16)]
        wo = hval[pl.ds(RPW + off, 16)]
        vs = lax.bitcast_convert_type(
            jnp.where((syi & 1) == 1, ws & np.int32(-65536),
                      jnp.left_shift(ws, 16)), jnp.float32)
        vo = lax.bitcast_convert_type(
            jnp.where((oyi & 1) == 1, wo & np.int32(-65536),
                      jnp.left_shift(wo, 16)), jnp.float32)
        sob[pl.ds(off, 16)] = vs * vo
        return 0

    lax.fori_loop(0, GROUPS, so_group, 0)

    for g in range(GROUPS):
        raf_cps[g].wait()
        reduce_group(g)

    pltpu.sync_copy(integb, integ_hbm.at[pl.ds(base, RPW)])
    pltpu.sync_copy(sob, so_hbm.at[pl.ds(base, RPW)])
    pltpu.sync_copy(validb, valid_hbm.at[pl.ds(base, RPW)])


_sc_compute = functools.partial(
    pl.kernel,
    out_type=(jax.ShapeDtypeStruct((R,), jnp.float32),
              jax.ShapeDtypeStruct((R,), jnp.float32),
              jax.ShapeDtypeStruct((R,), jnp.float32)),
    mesh=plsc.VectorSubcoreMesh(core_axis_name="c", subcore_axis_name="s",
                                num_cores=NC, num_subcores=NS),
    scratch_types=[
        pltpu.VMEM((8 * RPW,), jnp.int32),
        pltpu.VMEM((8192,), jnp.int32),
        pltpu.VMEM((8192,), jnp.int32),
        pltpu.VMEM((2 * RPW,), jnp.int32),
        pltpu.VMEM((2 * RPW,), jnp.int32),
        pltpu.VMEM((RPW,), jnp.float32),
        pltpu.VMEM((RPW,), jnp.float32),
        pltpu.VMEM((RPW,), jnp.float32),
        pltpu.VMEM((RPW,), jnp.float32),
        pltpu.VMEM((RPW,), jnp.float32),
        pltpu.SemaphoreType.DMA,
        pltpu.SemaphoreType.DMA,
        pltpu.SemaphoreType.DMA,
    ],
)(_sc_body)


def _loss_body(so_col, integ_bc, valid_col, valid_bc, out_ref):
    vbc = valid_bc[...]
    b_bc = jnp.where(vbc > np.float32(0.0),
                     jnp.log(integ_bc[...]), NEG_INF)

    def body(i, acc):
        so16 = so_col[pl.ds(i * 16, 16), :]
        mi = valid_col[pl.ds(i * 16, 16), :]
        a16 = jnp.where(mi > np.float32(0.0),
                        jnp.log(so16) - LOG_EPS, NEG_INF)
        return acc + jnp.maximum(a16 + b_bc, np.float32(0.0))

    acc = lax.fori_loop(0, R // 16, body,
                        jnp.zeros((16, R), jnp.float32))
    nv = jnp.sum(vbc) * np.float32(1.0 / 16.0)
    s = jnp.sum(acc) + LOG_EPS * nv * nv
    loss = -s / jnp.maximum(nv * nv, np.float32(1.0)) * LOSS_W
    out_ref[...] = loss.reshape(1, 1)


_loss_call = pl.pallas_call(
    _loss_body,
    out_shape=jax.ShapeDtypeStruct((1, 1), jnp.float32),
)


def kernel(rafs, heatmaps, batch_inds, subj_classes, obj_classes,
           subj_centers, obj_centers, predicates):
    raf_flat = _raf_pack(rafs).reshape(-1)
    hm_flat = _hm_pack(heatmaps).reshape(-1)
    integ, so, valid = _sc_compute(
        raf_flat, hm_flat, batch_inds, subj_classes, obj_classes, predicates,
        subj_centers[:, 0], subj_centers[:, 1],
        obj_centers[:, 0], obj_centers[:, 1])
    loss = _loss_call(so.reshape(R, 1),
                      jnp.broadcast_to(integ.reshape(1, R), (16, R)),
                      valid.reshape(R, 1),
                      jnp.broadcast_to(valid.reshape(1, R), (16, R)))
    return loss.reshape(())


# speedup vs baseline: 2.8801x; 1.1217x over previous
import functools

import jax
import jax.numpy as jnp
import numpy as np
from jax import lax
from jax.experimental import pallas as pl
from jax.experimental.pallas import tpu as pltpu
from jax.experimental.pallas import tpu_sc as plsc

B = 4
P = 50
H = 200
W = 200
C = 80
R = 2048
S = 128

NC = 2
NS = 16
NW = NC * NS
RPW = R // NW
GROUPS = RPW // 16
GSZ = 16 * S

PLANE = 400 * 128
HALF = 200 * 128

INV_T = np.float32(1.0 / (S - 1))
INV_S = np.float32(1.0 / S)
MAGIC = np.float32(2.0 ** 23)
RSQRT_MAGIC = np.int32(0x5F3759DF)
LOG_EPS = np.float32(np.log(np.float32(1e-12)))
NEG_INF = np.float32(-np.inf)
LOSS_W = np.float32(0.1)


RPP = 10
HPP = 16
HPLANE = 200 * 128
HHALF = 100 * 128


def _pack_body(raf_ref, hm_ref, rout_ref, hout_ref):
    for j in range(RPP):
        c0 = jnp.minimum(jnp.maximum(raf_ref[0, 2 * j], np.float32(-1.0)),
                         np.float32(1.0))
        c1 = jnp.minimum(jnp.maximum(raf_ref[0, 2 * j + 1], np.float32(-1.0)),
                         np.float32(1.0))
        b0 = lax.bitcast_convert_type(c0.astype(jnp.bfloat16), jnp.uint16)
        b1 = lax.bitcast_convert_type(c1.astype(jnp.bfloat16), jnp.uint16)
        u = jnp.left_shift(b1.astype(jnp.int32), 16) | b0.astype(jnp.int32)
        rout_ref[pl.ds(j * 400, 200), :] = u[:, 0:128]
        rout_ref[pl.ds(j * 400 + 200, 200), :] = u[:, 72:200]
    for j in range(HPP):
        v = hm_ref[0, j]
        bb = lax.bitcast_convert_type(v.astype(jnp.bfloat16), jnp.uint16)
        b3 = bb.reshape(100, 2, 200)
        be = b3[:, 0, :].astype(jnp.int32)
        bo = b3[:, 1, :].astype(jnp.int32)
        u = jnp.left_shift(bo, 16) | be
        hout_ref[pl.ds(j * 200, 100), :] = u[:, 0:128]
        hout_ref[pl.ds(j * 200 + 100, 100), :] = u[:, 72:200]


_pack = pl.pallas_call(
    _pack_body,
    grid=(B * P // RPP,),
    in_specs=[
        pl.BlockSpec((1, 2 * RPP, H, W),
                     lambda i: (i // (P // RPP), i % (P // RPP), 0, 0)),
        pl.BlockSpec((1, HPP, H, W),
                     lambda i: (i // (C // HPP), i % (C // HPP), 0, 0)),
    ],
    out_specs=(pl.BlockSpec((400 * RPP, 128), lambda i: (i, 0)),
               pl.BlockSpec((200 * HPP, 128), lambda i: (i, 0))),
    out_shape=(jax.ShapeDtypeStruct((B * P * 400, 128), jnp.int32),
               jax.ShapeDtypeStruct((B * C * 200, 128), jnp.int32)),
)


def _rsqrt_f32(x):
    i = lax.bitcast_convert_type(x, jnp.int32)
    i = RSQRT_MAGIC - lax.shift_right_logical(i, 1)
    y = lax.bitcast_convert_type(i, jnp.float32)
    for _ in range(3):
        y = y * (np.float32(1.5) - np.float32(0.5) * x * y * y)
    return y


def _rint_idx(x):
    r = (x + MAGIC) - MAGIC
    r = jnp.minimum(jnp.maximum(r, np.float32(0.0)), np.float32(199.0))
    return r.astype(jnp.int32)


def _lane_idx(p, q):
    return q * 128 + jnp.where(p < 128, p, HALF - 72 + p)


def _sc_body(raf_hbm, hm_hbm, bi_hbm, scl_hbm, ocl_hbm, prd_hbm,
             sx_hbm, sy_hbm, ox_hbm, oy_hbm,
             integ_hbm, so_hbm, valid_hbm,
             int_buf, idx_buf, g_buf, hidx, hval,
             uxb, uyb, integb, sob, validb, sem_in, sem_r, sem_h):
    wid = lax.axis_index("c") * NS + lax.axis_index("s")
    base = wid * RPW

    ins = (bi_hbm, scl_hbm, ocl_hbm, prd_hbm, sx_hbm, sy_hbm, ox_hbm, oy_hbm)
    cps = [pltpu.async_copy(src.at[pl.ds(base, RPW)],
                            int_buf.at[pl.ds(f * RPW, RPW)], sem_in)
           for f, src in enumerate(ins)]
    for cp in cps:
        cp.wait()

    def meta_group(g, _):
        off = g * 16
        bi = int_buf[pl.ds(0 * RPW + off, 16)]
        scl = int_buf[pl.ds(1 * RPW + off, 16)]
        ocl = int_buf[pl.ds(2 * RPW + off, 16)]
        sxi = int_buf[pl.ds(4 * RPW + off, 16)]
        syi = int_buf[pl.ds(5 * RPW + off, 16)]
        oxi = int_buf[pl.ds(6 * RPW + off, 16)]
        oyi = int_buf[pl.ds(7 * RPW + off, 16)]

        hidx[pl.ds(off, 16)] = ((bi * C + scl) * HPLANE +
                                lax.shift_right_logical(syi, 1) * 128 +
                                jnp.where(sxi < 128, sxi, HHALF - 72 + sxi))
        hidx[pl.ds(RPW + off, 16)] = ((bi * C + ocl) * HPLANE +
                                      lax.shift_right_logical(oyi, 1) * 128 +
                                      jnp.where(oxi < 128, oxi, HHALF - 72 + oxi))

        sxf = sxi.astype(jnp.float32)
        syf = syi.astype(jnp.float32)
        oxf = oxi.astype(jnp.float32)
        oyf = oyi.astype(jnp.float32)
        dx = oxf - sxf
        dy = oyf - syf
        n2 = dx * dx + dy * dy
        r = _rsqrt_f32(jnp.maximum(n2, np.float32(1.0)))
        uxb[pl.ds(off, 16)] = dx * r
        uyb[pl.ds(off, 16)] = dy * r
        validb[pl.ds(off, 16)] = jnp.where(n2 > np.float32(0.0),
                                           np.float32(1.0), np.float32(0.0))
        return 0

    lax.fori_loop(0, GROUPS, meta_group, 0)
    cp_h = pltpu.async_copy(hm_hbm.at[hidx], hval, sem_h)

    def build_group(g):
        off = g * 16
        bi = int_buf[pl.ds(0 * RPW + off, 16)]
        prd = int_buf[pl.ds(3 * RPW + off, 16)]
        sxf = int_buf[pl.ds(4 * RPW + off, 16)].astype(jnp.float32)
        syf = int_buf[pl.ds(5 * RPW + off, 16)].astype(jnp.float32)
        oxf = int_buf[pl.ds(6 * RPW + off, 16)].astype(jnp.float32)
        oyf = int_buf[pl.ds(7 * RPW + off, 16)].astype(jnp.float32)
        rbase = (bi * P + prd) * PLANE
        ddx = sxf - oxf
        ddy = syf - oyf

        def build_row(r32, _):
            rowoff = g * GSZ + r32 * 64
            for k in range(4):
                s = r32 * 4 + k
                t = s.astype(jnp.float32) * INV_T
                px = _rint_idx(oxf + t * ddx)
                py = _rint_idx(oyf + t * ddy)
                idx_buf[pl.ds(rowoff + k * 16, 16)] = rbase + _lane_idx(px, py)
            return 0

        lax.fori_loop(0, 32, build_row, 0)

    def reduce_group(g):
        off = g * 16
        ux = uxb[pl.ds(off, 16)]
        uy = uyb[pl.ds(off, 16)]

        def red_row(r32, acc):
            rowoff = g * GSZ + r32 * 64
            for k in range(4):
                v = g_buf[pl.ds(rowoff + k * 16, 16)]
                g1 = lax.bitcast_convert_type(
                    v & np.int32(-65536), jnp.float32)
                g0 = lax.bitcast_convert_type(
                    jnp.left_shift(v, 16), jnp.float32)
                acc = acc + g0 * ux + g1 * uy
            return acc

        acc = lax.fori_loop(0, 32, red_row, jnp.zeros((16,), jnp.float32))
        integb[pl.ds(off, 16)] = jnp.minimum(
            jnp.maximum(acc * INV_S, np.float32(0.0)), np.float32(1.0))

    raf_cps = []
    for g in range(GROUPS):
        build_group(g)
        raf_cps.append(pltpu.async_copy(
            raf_hbm.at[idx_buf.at[pl.ds(g * GSZ, GSZ)]],
            g_buf.at[pl.ds(g * GSZ, GSZ)], sem_r))

    cp_h.wait()

    def so_group(g, _):
        off = g * 16
        syi = int_buf[pl.ds(5 * RPW + off, 16)]
        oyi = int_buf[pl.ds(7 * RPW + off, 16)]
        ws = hval[pl.ds(off, 16)]
        wo = hval[pl.ds(RPW + off, 16)]
        vs = lax.bitcast_convert_type(
            jnp.where((syi & 1) == 1, ws & np.int32(-65536),
                      jnp.left_shift(ws, 16)), jnp.float32)
        vo = lax.bitcast_convert_type(
            jnp.where((oyi & 1) == 1, wo & np.int32(-65536),
                      jnp.left_shift(wo, 16)), jnp.float32)
        sob[pl.ds(off, 16)] = vs * vo
        return 0

    lax.fori_loop(0, GROUPS, so_group, 0)

    for g in range(GROUPS):
        raf_cps[g].wait()
        reduce_group(g)

    pltpu.sync_copy(integb, integ_hbm.at[pl.ds(base, RPW)])
    pltpu.sync_copy(sob, so_hbm.at[pl.ds(base, RPW)])
    pltpu.sync_copy(validb, valid_hbm.at[pl.ds(base, RPW)])


_sc_compute = functools.partial(
    pl.kernel,
    out_type=(jax.ShapeDtypeStruct((R,), jnp.float32),
              jax.ShapeDtypeStruct((R,), jnp.float32),
              jax.ShapeDtypeStruct((R,), jnp.float32)),
    mesh=plsc.VectorSubcoreMesh(core_axis_name="c", subcore_axis_name="s",
                                num_cores=NC, num_subcores=NS),
    scratch_types=[
        pltpu.VMEM((8 * RPW,), jnp.int32),
        pltpu.VMEM((8192,), jnp.int32),
        pltpu.VMEM((8192,), jnp.int32),
        pltpu.VMEM((2 * RPW,), jnp.int32),
        pltpu.VMEM((2 * RPW,), jnp.int32),
        pltpu.VMEM((RPW,), jnp.float32),
        pltpu.VMEM((RPW,), jnp.float32),
        pltpu.VMEM((RPW,), jnp.float32),
        pltpu.VMEM((RPW,), jnp.float32),
        pltpu.VMEM((RPW,), jnp.float32),
        pltpu.SemaphoreType.DMA,
        pltpu.SemaphoreType.DMA,
        pltpu.SemaphoreType.DMA,
    ],
)(_sc_body)


def _loss_body(so_col, integ_bc, valid_col, valid_bc, out_ref):
    vbc = valid_bc[...]
    b_bc = jnp.where(vbc > np.float32(0.0),
                     jnp.log(integ_bc[...]), NEG_INF)

    def body(i, acc):
        so16 = so_col[pl.ds(i * 16, 16), :]
        mi = valid_col[pl.ds(i * 16, 16), :]
        a16 = jnp.where(mi > np.float32(0.0),
                        jnp.log(so16) - LOG_EPS, NEG_INF)
        return acc + jnp.maximum(a16 + b_bc, np.float32(0.0))

    acc = lax.fori_loop(0, R // 16, body,
                        jnp.zeros((16, R), jnp.float32))
    nv = jnp.sum(vbc) * np.float32(1.0 / 16.0)
    s = jnp.sum(acc) + LOG_EPS * nv * nv
    loss = -s / jnp.maximum(nv * nv, np.float32(1.0)) * LOSS_W
    out_ref[...] = loss.reshape(1, 1)


_loss_call = pl.pallas_call(
    _loss_body,
    out_shape=jax.ShapeDtypeStruct((1, 1), jnp.float32),
)


def kernel(rafs, heatmaps, batch_inds, subj_classes, obj_classes,
           subj_centers, obj_centers, predicates):
    raf_packed, hm_packed = _pack(rafs, heatmaps)
    raf_flat = raf_packed.reshape(-1)
    hm_flat = hm_packed.reshape(-1)
    integ, so, valid = _sc_compute(
        raf_flat, hm_flat, batch_inds, subj_classes, obj_classes, predicates,
        subj_centers[:, 0], subj_centers[:, 1],
        obj_centers[:, 0], obj_centers[:, 1])
    loss = _loss_call(so.reshape(R, 1),
                      jnp.broadcast_to(integ.reshape(1, R), (16, R)),
                      valid.reshape(R, 1),
                      jnp.broadcast_to(valid.reshape(1, R), (16, R)))
    return loss.reshape(())


# speedup vs baseline: 2.9948x; 1.0398x over previous
import functools

import jax
import jax.numpy as jnp
import numpy as np
from jax import lax
from jax.experimental import pallas as pl
from jax.experimental.pallas import tpu as pltpu
from jax.experimental.pallas import tpu_sc as plsc

B = 4
P = 50
H = 200
W = 200
C = 80
R = 2048
S = 128

NC = 2
NS = 16
NW = NC * NS
RPW = R // NW
GROUPS = RPW // 16
GSZ = 16 * S

PLANE = 400 * 128
HALF = 200 * 128

INV_T = np.float32(1.0 / (S - 1))
INV_S = np.float32(1.0 / S)
MAGIC = np.float32(2.0 ** 23)
RSQRT_MAGIC = np.int32(0x5F3759DF)
LOG_EPS = np.float32(np.log(np.float32(1e-12)))
NEG_INF = np.float32(-np.inf)
LOSS_W = np.float32(0.1)


RPP = 10
HPP = 16
HPLANE = 200 * 128
HHALF = 100 * 128


def _pack_body(raf_ref, hm_ref, rout_ref, hout_ref):
    for j in range(RPP):
        c0 = jnp.minimum(jnp.maximum(raf_ref[0, 2 * j], np.float32(-1.0)),
                         np.float32(1.0))
        c1 = jnp.minimum(jnp.maximum(raf_ref[0, 2 * j + 1], np.float32(-1.0)),
                         np.float32(1.0))
        b0 = lax.bitcast_convert_type(c0.astype(jnp.bfloat16), jnp.uint16)
        b1 = lax.bitcast_convert_type(c1.astype(jnp.bfloat16), jnp.uint16)
        u = jnp.left_shift(b1.astype(jnp.int32), 16) | b0.astype(jnp.int32)
        rout_ref[pl.ds(j * 400, 200), :] = u[:, 0:128]
        rout_ref[pl.ds(j * 400 + 200, 200), :] = u[:, 72:200]
    for j in range(HPP):
        v = hm_ref[0, j]
        bb = lax.bitcast_convert_type(v.astype(jnp.bfloat16), jnp.uint16)
        b3 = bb.reshape(100, 2, 200)
        be = b3[:, 0, :].astype(jnp.int32)
        bo = b3[:, 1, :].astype(jnp.int32)
        u = jnp.left_shift(bo, 16) | be
        hout_ref[pl.ds(j * 200, 100), :] = u[:, 0:128]
        hout_ref[pl.ds(j * 200 + 100, 100), :] = u[:, 72:200]


_pack = pl.pallas_call(
    _pack_body,
    grid=(B * P // RPP,),
    in_specs=[
        pl.BlockSpec((1, 2 * RPP, H, W),
                     lambda i: (i // (P // RPP), i % (P // RPP), 0, 0)),
        pl.BlockSpec((1, HPP, H, W),
                     lambda i: (i // (C // HPP), i % (C // HPP), 0, 0)),
    ],
    out_specs=(pl.BlockSpec((400 * RPP, 128), lambda i: (i, 0)),
               pl.BlockSpec((200 * HPP, 128), lambda i: (i, 0))),
    out_shape=(jax.ShapeDtypeStruct((B * P * 400, 128), jnp.int32),
               jax.ShapeDtypeStruct((B * C * 200, 128), jnp.int32)),
)


def _rsqrt_f32(x):
    i = lax.bitcast_convert_type(x, jnp.int32)
    i = RSQRT_MAGIC - lax.shift_right_logical(i, 1)
    y = lax.bitcast_convert_type(i, jnp.float32)
    for _ in range(3):
        y = y * (np.float32(1.5) - np.float32(0.5) * x * y * y)
    return y


def _rint_idx(x):
    r = (x + MAGIC) - MAGIC
    r = jnp.minimum(jnp.maximum(r, np.float32(0.0)), np.float32(199.0))
    return r.astype(jnp.int32)


def _lane_idx(p, q):
    return q * 128 + jnp.where(p < 128, p, HALF - 72 + p)


def _sc_body(raf_hbm, hm_hbm, bi_hbm, scl_hbm, ocl_hbm, prd_hbm,
             sx_hbm, sy_hbm, ox_hbm, oy_hbm,
             integ_hbm, so_hbm, valid_hbm,
             int_buf, idx_buf, g_buf, hidx, hval,
             uxb, uyb, integb, sob, validb, sem_in, sem_r, sem_h):
    wid = lax.axis_index("c") * NS + lax.axis_index("s")
    base = wid * RPW

    ins = (bi_hbm, scl_hbm, ocl_hbm, prd_hbm, sx_hbm, sy_hbm, ox_hbm, oy_hbm)
    cps = [pltpu.async_copy(src.at[pl.ds(base, RPW)],
                            int_buf.at[pl.ds(f * RPW, RPW)], sem_in)
           for f, src in enumerate(ins)]
    for cp in cps:
        cp.wait()

    def meta_group(g, _):
        off = g * 16
        bi = int_buf[pl.ds(0 * RPW + off, 16)]
        scl = int_buf[pl.ds(1 * RPW + off, 16)]
        ocl = int_buf[pl.ds(2 * RPW + off, 16)]
        sxi = int_buf[pl.ds(4 * RPW + off, 16)]
        syi = int_buf[pl.ds(5 * RPW + off, 16)]
        oxi = int_buf[pl.ds(6 * RPW + off, 16)]
        oyi = int_buf[pl.ds(7 * RPW + off, 16)]

        hidx[pl.ds(off, 16)] = ((bi * C + scl) * HPLANE +
                                lax.shift_right_logical(syi, 1) * 128 +
                                jnp.where(sxi < 128, sxi, HHALF - 72 + sxi))
        hidx[pl.ds(RPW + off, 16)] = ((bi * C + ocl) * HPLANE +
                                      lax.shift_right_logical(oyi, 1) * 128 +
                                      jnp.where(oxi < 128, oxi, HHALF - 72 + oxi))

        sxf = sxi.astype(jnp.float32)
        syf = syi.astype(jnp.float32)
        oxf = oxi.astype(jnp.float32)
        oyf = oyi.astype(jnp.float32)
        dx = oxf - sxf
        dy = oyf - syf
        n2 = dx * dx + dy * dy
        r = _rsqrt_f32(jnp.maximum(n2, np.float32(1.0)))
        uxb[pl.ds(off, 16)] = dx * r
        uyb[pl.ds(off, 16)] = dy * r
        validb[pl.ds(off, 16)] = jnp.where(n2 > np.float32(0.0),
                                           np.float32(1.0), np.float32(0.0))
        return 0

    lax.fori_loop(0, GROUPS, meta_group, 0)
    cp_h = pltpu.async_copy(hm_hbm.at[hidx], hval, sem_h)

    def build_group(g):
        off = g * 16
        bi = int_buf[pl.ds(0 * RPW + off, 16)]
        prd = int_buf[pl.ds(3 * RPW + off, 16)]
        sxf = int_buf[pl.ds(4 * RPW + off, 16)].astype(jnp.float32)
        syf = int_buf[pl.ds(5 * RPW + off, 16)].astype(jnp.float32)
        oxf = int_buf[pl.ds(6 * RPW + off, 16)].astype(jnp.float32)
        oyf = int_buf[pl.ds(7 * RPW + off, 16)].astype(jnp.float32)
        rbase = (bi * P + prd) * PLANE
        ddx = sxf - oxf
        ddy = syf - oyf

        def build_row(r32, _):
            rowoff = g * GSZ + r32 * 64
            for k in range(4):
                s = r32 * 4 + k
                t = s.astype(jnp.float32) * INV_T
                px = _rint_idx(oxf + t * ddx)
                py = _rint_idx(oyf + t * ddy)
                idx_buf[pl.ds(rowoff + k * 16, 16)] = rbase + _lane_idx(px, py)
            return 0

        lax.fori_loop(0, 32, build_row, 0)

    def reduce_group(g):
        off = g * 16
        ux = uxb[pl.ds(off, 16)]
        uy = uyb[pl.ds(off, 16)]

        def red_row(r32, acc):
            rowoff = g * GSZ + r32 * 64
            for k in range(4):
                v = g_buf[pl.ds(rowoff + k * 16, 16)]
                g1 = lax.bitcast_convert_type(
                    v & np.int32(-65536), jnp.float32)
                g0 = lax.bitcast_convert_type(
                    jnp.left_shift(v, 16), jnp.float32)
                acc = acc + g0 * ux + g1 * uy
            return acc

        acc = lax.fori_loop(0, 32, red_row, jnp.zeros((16,), jnp.float32))
        integb[pl.ds(off, 16)] = jnp.minimum(
            jnp.maximum(acc * INV_S, np.float32(0.0)), np.float32(1.0))

    raf_cps = []
    for g in range(GROUPS):
        build_group(g)
        raf_cps.append(pltpu.async_copy(
            raf_hbm.at[idx_buf.at[pl.ds(g * GSZ, GSZ)]],
            g_buf.at[pl.ds(g * GSZ, GSZ)], sem_r))

    cp_h.wait()

    def so_group(g, _):
        off = g * 16
        syi = int_buf[pl.ds(5 * RPW + off, 16)]
        oyi = int_buf[pl.ds(7 * RPW + off, 16)]
        ws = hval[pl.ds(off, 16)]
        wo = hval[pl.ds(RPW + off, 16)]
        vs = lax.bitcast_convert_type(
            jnp.where((syi & 1) == 1, ws & np.int32(-65536),
                      jnp.left_shift(ws, 16)), jnp.float32)
        vo = lax.bitcast_convert_type(
            jnp.where((oyi & 1) == 1, wo & np.int32(-65536),
                      jnp.left_shift(wo, 16)), jnp.float32)
        sob[pl.ds(off, 16)] = vs * vo
        return 0

    lax.fori_loop(0, GROUPS, so_group, 0)

    for g in range(GROUPS):
        raf_cps[g].wait()
        reduce_group(g)

    pltpu.sync_copy(integb, integ_hbm.at[pl.ds(base, RPW)])
    pltpu.sync_copy(sob, so_hbm.at[pl.ds(base, RPW)])
    pltpu.sync_copy(validb, valid_hbm.at[pl.ds(base, RPW)])


_sc_compute = functools.partial(
    pl.kernel,
    out_type=(jax.ShapeDtypeStruct((R,), jnp.float32),
              jax.ShapeDtypeStruct((R,), jnp.float32),
              jax.ShapeDtypeStruct((R,), jnp.float32)),
    mesh=plsc.VectorSubcoreMesh(core_axis_name="c", subcore_axis_name="s",
                                num_cores=NC, num_subcores=NS),
    scratch_types=[
        pltpu.VMEM((8 * RPW,), jnp.int32),
        pltpu.VMEM((8192,), jnp.int32),
        pltpu.VMEM((8192,), jnp.int32),
        pltpu.VMEM((2 * RPW,), jnp.int32),
        pltpu.VMEM((2 * RPW,), jnp.int32),
        pltpu.VMEM((RPW,), jnp.float32),
        pltpu.VMEM((RPW,), jnp.float32),
        pltpu.VMEM((RPW,), jnp.float32),
        pltpu.VMEM((RPW,), jnp.float32),
        pltpu.VMEM((RPW,), jnp.float32),
        pltpu.SemaphoreType.DMA,
        pltpu.SemaphoreType.DMA,
        pltpu.SemaphoreType.DMA,
    ],
)(_sc_body)


def _loss_body(so_col, integ_bc, valid_col, valid_bc, out_ref):
    vbc = valid_bc[...]
    b_bc = jnp.where(vbc > np.float32(0.0),
                     jnp.log(integ_bc[...]), NEG_INF)

    def body(i, acc):
        t = acc
        for u in range(2):
            so16 = so_col[pl.ds((2 * i + u) * 16, 16), :]
            mi = valid_col[pl.ds((2 * i + u) * 16, 16), :]
            a16 = jnp.where(mi > np.float32(0.0),
                            jnp.log(so16) - LOG_EPS, NEG_INF)
            t = t + jnp.maximum(a16 + b_bc, np.float32(0.0))
        return t

    acc = lax.fori_loop(0, R // 32, body,
                        jnp.zeros((16, R), jnp.float32))
    nv = jnp.sum(vbc) * np.float32(1.0 / 16.0)
    s = jnp.sum(acc) + LOG_EPS * nv * nv
    loss = -s / jnp.maximum(nv * nv, np.float32(1.0)) * LOSS_W
    out_ref[...] = loss.reshape(1, 1)


_loss_call = pl.pallas_call(
    _loss_body,
    out_shape=jax.ShapeDtypeStruct((1, 1), jnp.float32),
)


def kernel(rafs, heatmaps, batch_inds, subj_classes, obj_classes,
           subj_centers, obj_centers, predicates):
    raf_packed, hm_packed = _pack(rafs, heatmaps)
    raf_flat = raf_packed.reshape(-1)
    hm_flat = hm_packed.reshape(-1)
    integ, so, valid = _sc_compute(
        raf_flat, hm_flat, batch_inds, subj_classes, obj_classes, predicates,
        subj_centers[:, 0], subj_centers[:, 1],
        obj_centers[:, 0], obj_centers[:, 1])
    loss = _loss_call(so.reshape(R, 1),
                      jnp.broadcast_to(integ.reshape(1, R), (16, R)),
                      valid.reshape(R, 1),
                      jnp.broadcast_to(valid.reshape(1, R), (16, R)))
    return loss.reshape(())


# speedup vs baseline: 3.1443x; 1.0499x over previous
import functools

import jax
import jax.numpy as jnp
import numpy as np
from jax import lax
from jax.experimental import pallas as pl
from jax.experimental.pallas import tpu as pltpu
from jax.experimental.pallas import tpu_sc as plsc

B = 4
P = 50
H = 200
W = 200
C = 80
R = 2048
S = 128

NC = 2
NS = 16
NW = NC * NS
RPW = R // NW
GROUPS = RPW // 16
GSZ = 16 * S

PLANE = 400 * 128
HALF = 200 * 128

INV_T = np.float32(1.0 / (S - 1))
INV_S = np.float32(1.0 / S)
MAGIC = np.float32(2.0 ** 23)
RSQRT_MAGIC = np.int32(0x5F3759DF)
LOG_EPS = np.float32(np.log(np.float32(1e-12)))
NEG_INF = np.float32(-np.inf)
LOSS_W = np.float32(0.1)


RPP = 10
HPP = 16
HPLANE = 200 * 128
HHALF = 100 * 128


def _pack_body(raf_ref, hm_ref, rout_ref, hout_ref):
    for j in range(RPP):
        c0 = jnp.minimum(jnp.maximum(raf_ref[0, 2 * j], np.float32(-1.0)),
                         np.float32(1.0))
        c1 = jnp.minimum(jnp.maximum(raf_ref[0, 2 * j + 1], np.float32(-1.0)),
                         np.float32(1.0))
        b0 = lax.bitcast_convert_type(c0.astype(jnp.bfloat16), jnp.uint16)
        b1 = lax.bitcast_convert_type(c1.astype(jnp.bfloat16), jnp.uint16)
        u = jnp.left_shift(b1.astype(jnp.int32), 16) | b0.astype(jnp.int32)
        rout_ref[pl.ds(j * 400, 200), :] = u[:, 0:128]
        rout_ref[pl.ds(j * 400 + 200, 200), :] = u[:, 72:200]
    for j in range(HPP):
        v = hm_ref[0, j]
        bb = lax.bitcast_convert_type(v.astype(jnp.bfloat16), jnp.uint16)
        b3 = bb.reshape(100, 2, 200)
        be = b3[:, 0, :].astype(jnp.int32)
        bo = b3[:, 1, :].astype(jnp.int32)
        u = jnp.left_shift(bo, 16) | be
        hout_ref[pl.ds(j * 200, 100), :] = u[:, 0:128]
        hout_ref[pl.ds(j * 200 + 100, 100), :] = u[:, 72:200]


_pack = pl.pallas_call(
    _pack_body,
    grid=(B * P // RPP,),
    in_specs=[
        pl.BlockSpec((1, 2 * RPP, H, W),
                     lambda i: (i // (P // RPP), i % (P // RPP), 0, 0)),
        pl.BlockSpec((1, HPP, H, W),
                     lambda i: (i // (C // HPP), i % (C // HPP), 0, 0)),
    ],
    out_specs=(pl.BlockSpec((400 * RPP, 128), lambda i: (i, 0)),
               pl.BlockSpec((200 * HPP, 128), lambda i: (i, 0))),
    out_shape=(jax.ShapeDtypeStruct((B * P * 400, 128), jnp.int32),
               jax.ShapeDtypeStruct((B * C * 200, 128), jnp.int32)),
)


def _rsqrt_f32(x):
    i = lax.bitcast_convert_type(x, jnp.int32)
    i = RSQRT_MAGIC - lax.shift_right_logical(i, 1)
    y = lax.bitcast_convert_type(i, jnp.float32)
    for _ in range(3):
        y = y * (np.float32(1.5) - np.float32(0.5) * x * y * y)
    return y


def _rint_idx(x):
    r = (x + MAGIC) - MAGIC
    r = jnp.minimum(jnp.maximum(r, np.float32(0.0)), np.float32(199.0))
    return r.astype(jnp.int32)


def _lane_idx(p, q):
    return q * 128 + jnp.where(p < 128, p, HALF - 72 + p)


def _sc_body(raf_hbm, hm_hbm, bi_hbm, scl_hbm, ocl_hbm, prd_hbm,
             sx_hbm, sy_hbm, ox_hbm, oy_hbm,
             integ_hbm, so_hbm, valid_hbm,
             int_buf, idx_buf, g_buf, hidx, hval,
             uxb, uyb, integb, sob, validb, sem_in, sem_r, sem_h):
    wid = lax.axis_index("c") * NS + lax.axis_index("s")
    base = wid * RPW

    ins = (bi_hbm, scl_hbm, ocl_hbm, prd_hbm, sx_hbm, sy_hbm, ox_hbm, oy_hbm)
    cps = [pltpu.async_copy(src.at[pl.ds(base, RPW)],
                            int_buf.at[pl.ds(f * RPW, RPW)], sem_in)
           for f, src in enumerate(ins)]
    for cp in cps:
        cp.wait()

    def meta_group(g, _):
        off = g * 16
        bi = int_buf[pl.ds(0 * RPW + off, 16)]
        scl = int_buf[pl.ds(1 * RPW + off, 16)]
        ocl = int_buf[pl.ds(2 * RPW + off, 16)]
        sxi = int_buf[pl.ds(4 * RPW + off, 16)]
        syi = int_buf[pl.ds(5 * RPW + off, 16)]
        oxi = int_buf[pl.ds(6 * RPW + off, 16)]
        oyi = int_buf[pl.ds(7 * RPW + off, 16)]

        hidx[pl.ds(off, 16)] = ((bi * C + scl) * HPLANE +
                                lax.shift_right_logical(syi, 1) * 128 +
                                jnp.where(sxi < 128, sxi, HHALF - 72 + sxi))
        hidx[pl.ds(RPW + off, 16)] = ((bi * C + ocl) * HPLANE +
                                      lax.shift_right_logical(oyi, 1) * 128 +
                                      jnp.where(oxi < 128, oxi, HHALF - 72 + oxi))

        sxf = sxi.astype(jnp.float32)
        syf = syi.astype(jnp.float32)
        oxf = oxi.astype(jnp.float32)
        oyf = oyi.astype(jnp.float32)
        dx = oxf - sxf
        dy = oyf - syf
        n2 = dx * dx + dy * dy
        r = _rsqrt_f32(jnp.maximum(n2, np.float32(1.0)))
        uxb[pl.ds(off, 16)] = dx * r
        uyb[pl.ds(off, 16)] = dy * r
        validb[pl.ds(off, 16)] = jnp.where(n2 > np.float32(0.0),
                                           np.float32(1.0), np.float32(0.0))
        return 0

    lax.fori_loop(0, GROUPS, meta_group, 0)
    cp_h = pltpu.async_copy(hm_hbm.at[hidx], hval, sem_h)

    def build_group(g):
        off = g * 16
        bi = int_buf[pl.ds(0 * RPW + off, 16)]
        prd = int_buf[pl.ds(3 * RPW + off, 16)]
        sxf = int_buf[pl.ds(4 * RPW + off, 16)].astype(jnp.float32)
        syf = int_buf[pl.ds(5 * RPW + off, 16)].astype(jnp.float32)
        oxf = int_buf[pl.ds(6 * RPW + off, 16)].astype(jnp.float32)
        oyf = int_buf[pl.ds(7 * RPW + off, 16)].astype(jnp.float32)
        rbase = (bi * P + prd) * PLANE
        ddx = sxf - oxf
        ddy = syf - oyf

        def build_row(r32, _):
            rowoff = g * GSZ + r32 * 64
            for k in range(4):
                s = r32 * 4 + k
                t = s.astype(jnp.float32) * INV_T
                px = _rint_idx(oxf + t * ddx)
                py = _rint_idx(oyf + t * ddy)
                idx_buf[pl.ds(rowoff + k * 16, 16)] = rbase + _lane_idx(px, py)
            return 0

        lax.fori_loop(0, 32, build_row, 0)

    def reduce_group(g):
        off = g * 16
        ux = uxb[pl.ds(off, 16)]
        uy = uyb[pl.ds(off, 16)]

        def red_row(r32, acc):
            rowoff = g * GSZ + r32 * 64
            for k in range(4):
                v = g_buf[pl.ds(rowoff + k * 16, 16)]
                g1 = lax.bitcast_convert_type(
                    v & np.int32(-65536), jnp.float32)
                g0 = lax.bitcast_convert_type(
                    jnp.left_shift(v, 16), jnp.float32)
                acc = acc + g0 * ux + g1 * uy
            return acc

        acc = lax.fori_loop(0, 32, red_row, jnp.zeros((16,), jnp.float32))
        integb[pl.ds(off, 16)] = jnp.minimum(
            jnp.maximum(acc * INV_S, np.float32(0.0)), np.float32(1.0))

    raf_cps = []
    for g in range(GROUPS):
        build_group(g)
        raf_cps.append(pltpu.async_copy(
            raf_hbm.at[idx_buf.at[pl.ds(g * GSZ, GSZ)]],
            g_buf.at[pl.ds(g * GSZ, GSZ)], sem_r))

    cp_h.wait()

    def so_group(g, _):
        off = g * 16
        syi = int_buf[pl.ds(5 * RPW + off, 16)]
        oyi = int_buf[pl.ds(7 * RPW + off, 16)]
        ws = hval[pl.ds(off, 16)]
        wo = hval[pl.ds(RPW + off, 16)]
        vs = lax.bitcast_convert_type(
            jnp.where((syi & 1) == 1, ws & np.int32(-65536),
                      jnp.left_shift(ws, 16)), jnp.float32)
        vo = lax.bitcast_convert_type(
            jnp.where((oyi & 1) == 1, wo & np.int32(-65536),
                      jnp.left_shift(wo, 16)), jnp.float32)
        sob[pl.ds(off, 16)] = vs * vo
        return 0

    lax.fori_loop(0, GROUPS, so_group, 0)

    for g in range(GROUPS):
        raf_cps[g].wait()
        reduce_group(g)

    pltpu.sync_copy(integb, integ_hbm.at[pl.ds(base, RPW)])
    pltpu.sync_copy(sob, so_hbm.at[pl.ds(base, RPW)])
    pltpu.sync_copy(validb, valid_hbm.at[pl.ds(base, RPW)])


_sc_compute = functools.partial(
    pl.kernel,
    out_type=(jax.ShapeDtypeStruct((R,), jnp.float32),
              jax.ShapeDtypeStruct((R,), jnp.float32),
              jax.ShapeDtypeStruct((R,), jnp.float32)),
    mesh=plsc.VectorSubcoreMesh(core_axis_name="c", subcore_axis_name="s",
                                num_cores=NC, num_subcores=NS),
    scratch_types=[
        pltpu.VMEM((8 * RPW,), jnp.int32),
        pltpu.VMEM((8192,), jnp.int32),
        pltpu.VMEM((8192,), jnp.int32),
        pltpu.VMEM((2 * RPW,), jnp.int32),
        pltpu.VMEM((2 * RPW,), jnp.int32),
        pltpu.VMEM((RPW,), jnp.float32),
        pltpu.VMEM((RPW,), jnp.float32),
        pltpu.VMEM((RPW,), jnp.float32),
        pltpu.VMEM((RPW,), jnp.float32),
        pltpu.VMEM((RPW,), jnp.float32),
        pltpu.SemaphoreType.DMA,
        pltpu.SemaphoreType.DMA,
        pltpu.SemaphoreType.DMA,
    ],
)(_sc_body)


UNROLL = 4


def _loss_body(so_col, integ_row, valid_col, valid_row, out_ref):
    mj = valid_row[...]
    b_row = jnp.where(mj > np.float32(0.0),
                      jnp.log(integ_row[...]), NEG_INF)
    b_bc = jnp.concatenate([b_row] * 16, axis=0)

    def body(i, acc):
        t = acc
        for u in range(UNROLL):
            so16 = so_col[pl.ds((UNROLL * i + u) * 16, 16), :]
            mi = valid_col[pl.ds((UNROLL * i + u) * 16, 16), :]
            a16 = jnp.where(mi > np.float32(0.0),
                            jnp.log(so16) - LOG_EPS, NEG_INF)
            t = t + jnp.maximum(a16 + b_bc, np.float32(0.0))
        return t

    acc = lax.fori_loop(0, R // (16 * UNROLL), body,
                        jnp.zeros((16, R), jnp.float32))
    nv = jnp.sum(mj)
    s = jnp.sum(acc) + LOG_EPS * nv * nv
    loss = -s / jnp.maximum(nv * nv, np.float32(1.0)) * LOSS_W
    out_ref[...] = loss.reshape(1, 1)


_loss_call = pl.pallas_call(
    _loss_body,
    out_shape=jax.ShapeDtypeStruct((1, 1), jnp.float32),
)


def kernel(rafs, heatmaps, batch_inds, subj_classes, obj_classes,
           subj_centers, obj_centers, predicates):
    raf_packed, hm_packed = _pack(rafs, heatmaps)
    raf_flat = raf_packed.reshape(-1)
    hm_flat = hm_packed.reshape(-1)
    integ, so, valid = _sc_compute(
        raf_flat, hm_flat, batch_inds, subj_classes, obj_classes, predicates,
        subj_centers[:, 0], subj_centers[:, 1],
        obj_centers[:, 0], obj_centers[:, 1])
    loss = _loss_call(so.reshape(R, 1), integ.reshape(1, R),
                      valid.reshape(R, 1), valid.reshape(1, R))
    return loss.reshape(())


# speedup vs baseline: 3.1672x; 1.0073x over previous
import functools

import jax
import jax.numpy as jnp
import numpy as np
from jax import lax
from jax.experimental import pallas as pl
from jax.experimental.pallas import tpu as pltpu
from jax.experimental.pallas import tpu_sc as plsc

B = 4
P = 50
H = 200
W = 200
C = 80
R = 2048
S = 128

NC = 2
NS = 16
NW = NC * NS
RPW = R // NW
GROUPS = RPW // 16
GSZ = 16 * S

PLANE = 400 * 128
HALF = 200 * 128

INV_T = np.float32(1.0 / (S - 1))
INV_S = np.float32(1.0 / S)
MAGIC = np.float32(2.0 ** 23)
RSQRT_MAGIC = np.int32(0x5F3759DF)
LOG_EPS = np.float32(np.log(np.float32(1e-12)))
NEG_INF = np.float32(-np.inf)
LOSS_W = np.float32(0.1)


RPP = 20
HPP = 32
HPLANE = 200 * 128
HHALF = 100 * 128


def _pack_body(raf_ref, hm_ref, rout_ref, hout_ref):
    for j in range(RPP):
        c0 = jnp.minimum(jnp.maximum(raf_ref[0, 2 * j], np.float32(-1.0)),
                         np.float32(1.0))
        c1 = jnp.minimum(jnp.maximum(raf_ref[0, 2 * j + 1], np.float32(-1.0)),
                         np.float32(1.0))
        b0 = lax.bitcast_convert_type(c0.astype(jnp.bfloat16), jnp.uint16)
        b1 = lax.bitcast_convert_type(c1.astype(jnp.bfloat16), jnp.uint16)
        u = jnp.left_shift(b1.astype(jnp.int32), 16) | b0.astype(jnp.int32)
        rout_ref[pl.ds(j * 400, 200), :] = u[:, 0:128]
        rout_ref[pl.ds(j * 400 + 200, 200), :] = u[:, 72:200]
    for j in range(HPP):
        v = hm_ref[0, j]
        bb = lax.bitcast_convert_type(v.astype(jnp.bfloat16), jnp.uint16)
        b3 = bb.reshape(100, 2, 200)
        be = b3[:, 0, :].astype(jnp.int32)
        bo = b3[:, 1, :].astype(jnp.int32)
        u = jnp.left_shift(bo, 16) | be
        hout_ref[pl.ds(j * 200, 100), :] = u[:, 0:128]
        hout_ref[pl.ds(j * 200 + 100, 100), :] = u[:, 72:200]


_pack = pl.pallas_call(
    _pack_body,
    grid=(B * P // RPP,),
    in_specs=[
        pl.BlockSpec((1, 2 * RPP, H, W),
                     lambda i: (i // (P // RPP), i % (P // RPP), 0, 0)),
        pl.BlockSpec((1, HPP, H, W),
                     lambda i: (i // (C // HPP), i % (C // HPP), 0, 0)),
    ],
    out_specs=(pl.BlockSpec((400 * RPP, 128), lambda i: (i, 0)),
               pl.BlockSpec((200 * HPP, 128), lambda i: (i, 0))),
    out_shape=(jax.ShapeDtypeStruct((B * P * 400, 128), jnp.int32),
               jax.ShapeDtypeStruct((B * C * 200, 128), jnp.int32)),
)


def _rsqrt_f32(x):
    i = lax.bitcast_convert_type(x, jnp.int32)
    i = RSQRT_MAGIC - lax.shift_right_logical(i, 1)
    y = lax.bitcast_convert_type(i, jnp.float32)
    for _ in range(3):
        y = y * (np.float32(1.5) - np.float32(0.5) * x * y * y)
    return y


def _rint_idx(x):
    r = (x + MAGIC) - MAGIC
    r = jnp.minimum(jnp.maximum(r, np.float32(0.0)), np.float32(199.0))
    return r.astype(jnp.int32)


def _lane_idx(p, q):
    return q * 128 + jnp.where(p < 128, p, HALF - 72 + p)


def _sc_body(raf_hbm, hm_hbm, bi_hbm, scl_hbm, ocl_hbm, prd_hbm,
             sx_hbm, sy_hbm, ox_hbm, oy_hbm,
             integ_hbm, so_hbm, valid_hbm,
             int_buf, idx_buf, g_buf, hidx, hval,
             uxb, uyb, integb, sob, validb, sem_in, sem_r, sem_h):
    wid = lax.axis_index("c") * NS + lax.axis_index("s")
    base = wid * RPW

    ins = (bi_hbm, scl_hbm, ocl_hbm, prd_hbm, sx_hbm, sy_hbm, ox_hbm, oy_hbm)
    cps = [pltpu.async_copy(src.at[pl.ds(base, RPW)],
                            int_buf.at[pl.ds(f * RPW, RPW)], sem_in)
           for f, src in enumerate(ins)]
    for cp in cps:
        cp.wait()

    def meta_group(g, _):
        off = g * 16
        bi = int_buf[pl.ds(0 * RPW + off, 16)]
        scl = int_buf[pl.ds(1 * RPW + off, 16)]
        ocl = int_buf[pl.ds(2 * RPW + off, 16)]
        sxi = int_buf[pl.ds(4 * RPW + off, 16)]
        syi = int_buf[pl.ds(5 * RPW + off, 16)]
        oxi = int_buf[pl.ds(6 * RPW + off, 16)]
        oyi = int_buf[pl.ds(7 * RPW + off, 16)]

        hidx[pl.ds(off, 16)] = ((bi * C + scl) * HPLANE +
                                lax.shift_right_logical(syi, 1) * 128 +
                                jnp.where(sxi < 128, sxi, HHALF - 72 + sxi))
        hidx[pl.ds(RPW + off, 16)] = ((bi * C + ocl) * HPLANE +
                                      lax.shift_right_logical(oyi, 1) * 128 +
                                      jnp.where(oxi < 128, oxi, HHALF - 72 + oxi))

        sxf = sxi.astype(jnp.float32)
        syf = syi.astype(jnp.float32)
        oxf = oxi.astype(jnp.float32)
        oyf = oyi.astype(jnp.float32)
        dx = oxf - sxf
        dy = oyf - syf
        n2 = dx * dx + dy * dy
        r = _rsqrt_f32(jnp.maximum(n2, np.float32(1.0)))
        uxb[pl.ds(off, 16)] = dx * r
        uyb[pl.ds(off, 16)] = dy * r
        validb[pl.ds(off, 16)] = jnp.where(n2 > np.float32(0.0),
                                           np.float32(1.0), np.float32(0.0))
        return 0

    lax.fori_loop(0, GROUPS, meta_group, 0)
    cp_h = pltpu.async_copy(hm_hbm.at[hidx], hval, sem_h)

    def build_group(g):
        off = g * 16
        bi = int_buf[pl.ds(0 * RPW + off, 16)]
        prd = int_buf[pl.ds(3 * RPW + off, 16)]
        sxf = int_buf[pl.ds(4 * RPW + off, 16)].astype(jnp.float32)
        syf = int_buf[pl.ds(5 * RPW + off, 16)].astype(jnp.float32)
        oxf = int_buf[pl.ds(6 * RPW + off, 16)].astype(jnp.float32)
        oyf = int_buf[pl.ds(7 * RPW + off, 16)].astype(jnp.float32)
        rbase = (bi * P + prd) * PLANE
        ddx = sxf - oxf
        ddy = syf - oyf

        def build_row(r32, _):
            rowoff = g * GSZ + r32 * 64
            for k in range(4):
                s = r32 * 4 + k
                t = s.astype(jnp.float32) * INV_T
                px = _rint_idx(oxf + t * ddx)
                py = _rint_idx(oyf + t * ddy)
                idx_buf[pl.ds(rowoff + k * 16, 16)] = rbase + _lane_idx(px, py)
            return 0

        lax.fori_loop(0, 32, build_row, 0)

    def reduce_group(g):
        off = g * 16
        ux = uxb[pl.ds(off, 16)]
        uy = uyb[pl.ds(off, 16)]

        def red_row(r32, acc):
            rowoff = g * GSZ + r32 * 64
            for k in range(4):
                v = g_buf[pl.ds(rowoff + k * 16, 16)]
                g1 = lax.bitcast_convert_type(
                    v & np.int32(-65536), jnp.float32)
                g0 = lax.bitcast_convert_type(
                    jnp.left_shift(v, 16), jnp.float32)
                acc = acc + g0 * ux + g1 * uy
            return acc

        acc = lax.fori_loop(0, 32, red_row, jnp.zeros((16,), jnp.float32))
        integb[pl.ds(off, 16)] = jnp.minimum(
            jnp.maximum(acc * INV_S, np.float32(0.0)), np.float32(1.0))

    raf_cps = []
    for g in range(GROUPS):
        build_group(g)
        raf_cps.append(pltpu.async_copy(
            raf_hbm.at[idx_buf.at[pl.ds(g * GSZ, GSZ)]],
            g_buf.at[pl.ds(g * GSZ, GSZ)], sem_r))

    cp_h.wait()

    def so_group(g, _):
        off = g * 16
        syi = int_buf[pl.ds(5 * RPW + off, 16)]
        oyi = int_buf[pl.ds(7 * RPW + off, 16)]
        ws = hval[pl.ds(off, 16)]
        wo = hval[pl.ds(RPW + off, 16)]
        vs = lax.bitcast_convert_type(
            jnp.where((syi & 1) == 1, ws & np.int32(-65536),
                      jnp.left_shift(ws, 16)), jnp.float32)
        vo = lax.bitcast_convert_type(
            jnp.where((oyi & 1) == 1, wo & np.int32(-65536),
                      jnp.left_shift(wo, 16)), jnp.float32)
        sob[pl.ds(off, 16)] = vs * vo
        return 0

    lax.fori_loop(0, GROUPS, so_group, 0)

    for g in range(GROUPS):
        raf_cps[g].wait()
        reduce_group(g)

    pltpu.sync_copy(integb, integ_hbm.at[pl.ds(base, RPW)])
    pltpu.sync_copy(sob, so_hbm.at[pl.ds(base, RPW)])
    pltpu.sync_copy(validb, valid_hbm.at[pl.ds(base, RPW)])


_sc_compute = functools.partial(
    pl.kernel,
    out_type=(jax.ShapeDtypeStruct((R,), jnp.float32),
              jax.ShapeDtypeStruct((R,), jnp.float32),
              jax.ShapeDtypeStruct((R,), jnp.float32)),
    mesh=plsc.VectorSubcoreMesh(core_axis_name="c", subcore_axis_name="s",
                                num_cores=NC, num_subcores=NS),
    scratch_types=[
        pltpu.VMEM((8 * RPW,), jnp.int32),
        pltpu.VMEM((8192,), jnp.int32),
        pltpu.VMEM((8192,), jnp.int32),
        pltpu.VMEM((2 * RPW,), jnp.int32),
        pltpu.VMEM((2 * RPW,), jnp.int32),
        pltpu.VMEM((RPW,), jnp.float32),
        pltpu.VMEM((RPW,), jnp.float32),
        pltpu.VMEM((RPW,), jnp.float32),
        pltpu.VMEM((RPW,), jnp.float32),
        pltpu.VMEM((RPW,), jnp.float32),
        pltpu.SemaphoreType.DMA,
        pltpu.SemaphoreType.DMA,
        pltpu.SemaphoreType.DMA,
    ],
)(_sc_body)


UNROLL = 4


def _loss_body(so_col, integ_row, valid_col, valid_row, out_ref):
    mj = valid_row[...]
    b_row = jnp.where(mj > np.float32(0.0),
                      jnp.log(integ_row[...]), NEG_INF)
    b_bc = jnp.concatenate([b_row] * 16, axis=0)

    def body(i, acc):
        t = acc
        for u in range(UNROLL):
            so16 = so_col[pl.ds((UNROLL * i + u) * 16, 16), :]
            mi = valid_col[pl.ds((UNROLL * i + u) * 16, 16), :]
            a16 = jnp.where(mi > np.float32(0.0),
                            jnp.log(so16) - LOG_EPS, NEG_INF)
            t = t + jnp.maximum(a16 + b_bc, np.float32(0.0))
        return t

    acc = lax.fori_loop(0, R // (16 * UNROLL), body,
                        jnp.zeros((16, R), jnp.float32))
    nv = jnp.sum(mj)
    s = jnp.sum(acc) + LOG_EPS * nv * nv
    loss = -s / jnp.maximum(nv * nv, np.float32(1.0)) * LOSS_W
    out_ref[...] = loss.reshape(1, 1)


_loss_call = pl.pallas_call(
    _loss_body,
    out_shape=jax.ShapeDtypeStruct((1, 1), jnp.float32),
)


def kernel(rafs, heatmaps, batch_inds, subj_classes, obj_classes,
           subj_centers, obj_centers, predicates):
    raf_packed, hm_packed = _pack(rafs, heatmaps)
    raf_flat = raf_packed.reshape(-1)
    hm_flat = hm_packed.reshape(-1)
    integ, so, valid = _sc_compute(
        raf_flat, hm_flat, batch_inds, subj_classes, obj_classes, predicates,
        subj_centers[:, 0], subj_centers[:, 1],
        obj_centers[:, 0], obj_centers[:, 1])
    loss = _loss_call(so.reshape(R, 1), integ.reshape(1, R),
                      valid.reshape(R, 1), valid.reshape(1, R))
    return loss.reshape(())


# speedup vs baseline: 3.2201x; 1.0167x over previous
import functools

import jax
import jax.numpy as jnp
import numpy as np
from jax import lax
from jax.experimental import pallas as pl
from jax.experimental.pallas import tpu as pltpu
from jax.experimental.pallas import tpu_sc as plsc

B = 4
P = 50
H = 200
W = 200
C = 80
R = 2048
S = 128

NC = 2
NS = 16
NW = NC * NS
RPW = R // NW
GROUPS = RPW // 16
GSZ = 16 * S

PLANE = 400 * 128
HALF = 200 * 128

INV_T = np.float32(1.0 / (S - 1))
INV_S = np.float32(1.0 / S)
MAGIC = np.float32(2.0 ** 23)
RSQRT_MAGIC = np.int32(0x5F3759DF)
LOG_EPS = np.float32(np.log(np.float32(1e-12)))
NEG_INF = np.float32(-np.inf)
LOSS_W = np.float32(0.1)


RPP = 20
HPP = 32
HPLANE = 200 * 128
HHALF = 100 * 128


def _pack_body(raf_ref, hm_ref, rout_ref, hout_ref):
    for j in range(RPP):
        c0 = jnp.minimum(jnp.maximum(raf_ref[0, 2 * j], np.float32(-1.0)),
                         np.float32(1.0))
        c1 = jnp.minimum(jnp.maximum(raf_ref[0, 2 * j + 1], np.float32(-1.0)),
                         np.float32(1.0))
        b0 = lax.bitcast_convert_type(c0.astype(jnp.bfloat16), jnp.uint16)
        b1 = lax.bitcast_convert_type(c1.astype(jnp.bfloat16), jnp.uint16)
        u = jnp.left_shift(b1.astype(jnp.int32), 16) | b0.astype(jnp.int32)
        rout_ref[pl.ds(j * 400, 200), :] = u[:, 0:128]
        rout_ref[pl.ds(j * 400 + 200, 200), :] = u[:, 72:200]
    for j in range(HPP):
        v = hm_ref[0, j]
        bb = lax.bitcast_convert_type(v.astype(jnp.bfloat16), jnp.uint16)
        b3 = bb.reshape(100, 2, 200)
        be = b3[:, 0, :].astype(jnp.int32)
        bo = b3[:, 1, :].astype(jnp.int32)
        u = jnp.left_shift(bo, 16) | be
        hout_ref[pl.ds(j * 200, 100), :] = u[:, 0:128]
        hout_ref[pl.ds(j * 200 + 100, 100), :] = u[:, 72:200]


_pack = pl.pallas_call(
    _pack_body,
    grid=(B * P // RPP,),
    in_specs=[
        pl.BlockSpec((1, 2 * RPP, H, W),
                     lambda i: (i // (P // RPP), i % (P // RPP), 0, 0)),
        pl.BlockSpec((1, HPP, H, W),
                     lambda i: (i // (C // HPP), i % (C // HPP), 0, 0)),
    ],
    out_specs=(pl.BlockSpec((400 * RPP, 128), lambda i: (i, 0)),
               pl.BlockSpec((200 * HPP, 128), lambda i: (i, 0))),
    out_shape=(jax.ShapeDtypeStruct((B * P * 400, 128), jnp.int32),
               jax.ShapeDtypeStruct((B * C * 200, 128), jnp.int32)),
)


def _rsqrt_f32(x):
    i = lax.bitcast_convert_type(x, jnp.int32)
    i = RSQRT_MAGIC - lax.shift_right_logical(i, 1)
    y = lax.bitcast_convert_type(i, jnp.float32)
    for _ in range(3):
        y = y * (np.float32(1.5) - np.float32(0.5) * x * y * y)
    return y


def _rint_idx(x):
    r = (x + MAGIC) - MAGIC
    r = jnp.minimum(jnp.maximum(r, np.float32(0.0)), np.float32(199.0))
    return r.astype(jnp.int32)


def _lane_idx(p, q):
    return q * 128 + jnp.where(p < 128, p, HALF - 72 + p)


def _sc_body(raf_hbm, hm_hbm, bi_hbm, scl_hbm, ocl_hbm, prd_hbm,
             sx_hbm, sy_hbm, ox_hbm, oy_hbm,
             integ_hbm, so_hbm, valid_hbm,
             int_buf, idx_buf, g_buf, hidx, hval,
             uxb, uyb, integb, sob, validb, sem_in, sem_r, sem_h):
    wid = lax.axis_index("c") * NS + lax.axis_index("s")
    base = wid * RPW

    ins = (bi_hbm, scl_hbm, ocl_hbm, prd_hbm, sx_hbm, sy_hbm, ox_hbm, oy_hbm)
    cps = [pltpu.async_copy(src.at[pl.ds(base, RPW)],
                            int_buf.at[pl.ds(f * RPW, RPW)], sem_in)
           for f, src in enumerate(ins)]
    for cp in cps:
        cp.wait()

    def meta_group(g, _):
        off = g * 16
        bi = int_buf[pl.ds(0 * RPW + off, 16)]
        scl = int_buf[pl.ds(1 * RPW + off, 16)]
        ocl = int_buf[pl.ds(2 * RPW + off, 16)]
        sxi = int_buf[pl.ds(4 * RPW + off, 16)]
        syi = int_buf[pl.ds(5 * RPW + off, 16)]
        oxi = int_buf[pl.ds(6 * RPW + off, 16)]
        oyi = int_buf[pl.ds(7 * RPW + off, 16)]

        hidx[pl.ds(off, 16)] = ((bi * C + scl) * HPLANE +
                                lax.shift_right_logical(syi, 1) * 128 +
                                jnp.where(sxi < 128, sxi, HHALF - 72 + sxi))
        hidx[pl.ds(RPW + off, 16)] = ((bi * C + ocl) * HPLANE +
                                      lax.shift_right_logical(oyi, 1) * 128 +
                                      jnp.where(oxi < 128, oxi, HHALF - 72 + oxi))

        sxf = sxi.astype(jnp.float32)
        syf = syi.astype(jnp.float32)
        oxf = oxi.astype(jnp.float32)
        oyf = oyi.astype(jnp.float32)
        dx = oxf - sxf
        dy = oyf - syf
        n2 = dx * dx + dy * dy
        r = _rsqrt_f32(jnp.maximum(n2, np.float32(1.0)))
        uxb[pl.ds(off, 16)] = dx * r
        uyb[pl.ds(off, 16)] = dy * r
        validb[pl.ds(off, 16)] = jnp.where(n2 > np.float32(0.0),
                                           np.float32(1.0), np.float32(0.0))
        return 0

    lax.fori_loop(0, GROUPS, meta_group, 0)
    cp_h = pltpu.async_copy(hm_hbm.at[hidx], hval, sem_h)

    def build_group(g):
        off = g * 16
        bi = int_buf[pl.ds(0 * RPW + off, 16)]
        prd = int_buf[pl.ds(3 * RPW + off, 16)]
        sxf = int_buf[pl.ds(4 * RPW + off, 16)].astype(jnp.float32)
        syf = int_buf[pl.ds(5 * RPW + off, 16)].astype(jnp.float32)
        oxf = int_buf[pl.ds(6 * RPW + off, 16)].astype(jnp.float32)
        oyf = int_buf[pl.ds(7 * RPW + off, 16)].astype(jnp.float32)
        rbase = (bi * P + prd) * PLANE
        ddx = sxf - oxf
        ddy = syf - oyf

        def build_row(r32, _):
            rowoff = g * GSZ + r32 * 64
            for k in range(4):
                s = r32 * 4 + k
                t = s.astype(jnp.float32) * INV_T
                px = _rint_idx(oxf + t * ddx)
                py = _rint_idx(oyf + t * ddy)
                idx_buf[pl.ds(rowoff + k * 16, 16)] = rbase + _lane_idx(px, py)
            return 0

        lax.fori_loop(0, 32, build_row, 0)

    def reduce_group(g):
        off = g * 16
        ux = uxb[pl.ds(off, 16)]
        uy = uyb[pl.ds(off, 16)]

        def red_row(r32, acc):
            rowoff = g * GSZ + r32 * 64
            for k in range(4):
                v = g_buf[pl.ds(rowoff + k * 16, 16)]
                g1 = lax.bitcast_convert_type(
                    v & np.int32(-65536), jnp.float32)
                g0 = lax.bitcast_convert_type(
                    jnp.left_shift(v, 16), jnp.float32)
                acc = acc + g0 * ux + g1 * uy
            return acc

        acc = lax.fori_loop(0, 32, red_row, jnp.zeros((16,), jnp.float32))
        integb[pl.ds(off, 16)] = jnp.minimum(
            jnp.maximum(acc * INV_S, np.float32(0.0)), np.float32(1.0))

    raf_cps = []
    for g in range(GROUPS):
        build_group(g)
        raf_cps.append(pltpu.async_copy(
            raf_hbm.at[idx_buf.at[pl.ds(g * GSZ, GSZ)]],
            g_buf.at[pl.ds(g * GSZ, GSZ)], sem_r))

    cp_h.wait()

    def so_group(g, _):
        off = g * 16
        syi = int_buf[pl.ds(5 * RPW + off, 16)]
        oyi = int_buf[pl.ds(7 * RPW + off, 16)]
        ws = hval[pl.ds(off, 16)]
        wo = hval[pl.ds(RPW + off, 16)]
        vs = lax.bitcast_convert_type(
            jnp.where((syi & 1) == 1, ws & np.int32(-65536),
                      jnp.left_shift(ws, 16)), jnp.float32)
        vo = lax.bitcast_convert_type(
            jnp.where((oyi & 1) == 1, wo & np.int32(-65536),
                      jnp.left_shift(wo, 16)), jnp.float32)
        sob[pl.ds(off, 16)] = vs * vo
        return 0

    lax.fori_loop(0, GROUPS, so_group, 0)

    for g in range(GROUPS):
        raf_cps[g].wait()
        reduce_group(g)

    pltpu.sync_copy(integb, integ_hbm.at[pl.ds(base, RPW)])
    pltpu.sync_copy(sob, so_hbm.at[pl.ds(base, RPW)])
    pltpu.sync_copy(validb, valid_hbm.at[pl.ds(base, RPW)])


_sc_compute = functools.partial(
    pl.kernel,
    out_type=(jax.ShapeDtypeStruct((R,), jnp.float32),
              jax.ShapeDtypeStruct((R,), jnp.float32),
              jax.ShapeDtypeStruct((R,), jnp.float32)),
    mesh=plsc.VectorSubcoreMesh(core_axis_name="c", subcore_axis_name="s",
                                num_cores=NC, num_subcores=NS),
    scratch_types=[
        pltpu.VMEM((8 * RPW,), jnp.int32),
        pltpu.VMEM((8192,), jnp.int32),
        pltpu.VMEM((8192,), jnp.int32),
        pltpu.VMEM((2 * RPW,), jnp.int32),
        pltpu.VMEM((2 * RPW,), jnp.int32),
        pltpu.VMEM((RPW,), jnp.float32),
        pltpu.VMEM((RPW,), jnp.float32),
        pltpu.VMEM((RPW,), jnp.float32),
        pltpu.VMEM((RPW,), jnp.float32),
        pltpu.VMEM((RPW,), jnp.float32),
        pltpu.SemaphoreType.DMA,
        pltpu.SemaphoreType.DMA,
        pltpu.SemaphoreType.DMA,
    ],
)(_sc_body)


UNROLL = 8


def _loss_body(so_col, integ_row, valid_col, valid_row, out_ref):
    mj = valid_row[...]
    b_row = jnp.where(mj > np.float32(0.0),
                      jnp.log(integ_row[...]), NEG_INF)
    b_bc = jnp.concatenate([b_row] * 16, axis=0)

    def body(i, acc):
        t = acc
        for u in range(UNROLL):
            so16 = so_col[pl.ds((UNROLL * i + u) * 16, 16), :]
            mi = valid_col[pl.ds((UNROLL * i + u) * 16, 16), :]
            a16 = jnp.where(mi > np.float32(0.0),
                            jnp.log(so16) - LOG_EPS, NEG_INF)
            t = t + jnp.maximum(a16 + b_bc, np.float32(0.0))
        return t

    acc = lax.fori_loop(0, R // (16 * UNROLL), body,
                        jnp.zeros((16, R), jnp.float32))
    nv = jnp.sum(mj)
    s = jnp.sum(acc) + LOG_EPS * nv * nv
    loss = -s / jnp.maximum(nv * nv, np.float32(1.0)) * LOSS_W
    out_ref[...] = loss.reshape(1, 1)


_loss_call = pl.pallas_call(
    _loss_body,
    out_shape=jax.ShapeDtypeStruct((1, 1), jnp.float32),
)


def kernel(rafs, heatmaps, batch_inds, subj_classes, obj_classes,
           subj_centers, obj_centers, predicates):
    raf_packed, hm_packed = _pack(rafs, heatmaps)
    raf_flat = raf_packed.reshape(-1)
    hm_flat = hm_packed.reshape(-1)
    integ, so, valid = _sc_compute(
        raf_flat, hm_flat, batch_inds, subj_classes, obj_classes, predicates,
        subj_centers[:, 0], subj_centers[:, 1],
        obj_centers[:, 0], obj_centers[:, 1])
    loss = _loss_call(so.reshape(R, 1), integ.reshape(1, R),
                      valid.reshape(R, 1), valid.reshape(1, R))
    return loss.reshape(())
